# Initial kernel scaffold; baseline (speedup 1.0000x reference)
import jax
import jax.numpy as jnp
from jax.experimental import pallas as pl


def kernel(x, edge_index, W_l1, b_l1, W_r1, W_l2, b_l2, W_r2, g1, be1, g2, be2, Wm1, bm1, Wm2, bm2):
    raise NotImplementedError("write your pallas kernel here")


# speedup vs baseline: 1.0229x; 1.0229x over previous
import functools

import jax
import jax.numpy as jnp
from jax.experimental import pallas as pl
from jax.experimental.pallas import tpu as pltpu

N = 10000
D_IN = 128
D_H = 64
EPS = 1e-5

_ROWS = 1024


def _dense1_body(agg_ref, x_ref, wl_ref, wr_ref, bl_ref, g_ref, be_ref, o_ref):
    agg = agg_ref[...]
    x = x_ref[...]
    h = (jnp.dot(agg, wl_ref[...], preferred_element_type=jnp.float32)
         + jnp.dot(x, wr_ref[...], preferred_element_type=jnp.float32)
         + bl_ref[...])
    mu = jnp.mean(h, axis=-1, keepdims=True)
    var = jnp.mean((h - mu) ** 2, axis=-1, keepdims=True)
    h = (h - mu) * jax.lax.rsqrt(var + EPS) * g_ref[...] + be_ref[...]
    o_ref[...] = jnp.maximum(h, 0.0)


def _dense2_body(agg_ref, h_ref, wl_ref, wr_ref, bl_ref, g_ref, be_ref,
                 wm1_ref, bm1_ref, wm2_ref, bm2_ref, o_ref):
    agg = agg_ref[...]
    hp = h_ref[...]
    h = (jnp.dot(agg, wl_ref[...], preferred_element_type=jnp.float32)
         + jnp.dot(hp, wr_ref[...], preferred_element_type=jnp.float32)
         + bl_ref[...])
    mu = jnp.mean(h, axis=-1, keepdims=True)
    var = jnp.mean((h - mu) ** 2, axis=-1, keepdims=True)
    h = (h - mu) * jax.lax.rsqrt(var + EPS) * g_ref[...] + be_ref[...]
    h = jnp.maximum(h, 0.0)
    m = jnp.maximum(jnp.dot(h, wm1_ref[...], preferred_element_type=jnp.float32)
                    + bm1_ref[...], 0.0)
    z = jnp.sum(m * wm2_ref[...], axis=-1, keepdims=True) + bm2_ref[...]
    o_ref[...] = jax.nn.sigmoid(z)


def _full(shape):
    return pl.BlockSpec(shape, lambda i: tuple(0 for _ in shape))


def _dense1(agg, x, wlT, wrT, bl, g, be):
    grid = (pl.cdiv(N, _ROWS),)
    return pl.pallas_call(
        _dense1_body,
        grid=grid,
        in_specs=[
            pl.BlockSpec((_ROWS, D_IN), lambda i: (i, 0)),
            pl.BlockSpec((_ROWS, D_IN), lambda i: (i, 0)),
            _full((D_IN, D_H)), _full((D_IN, D_H)),
            _full((1, D_H)), _full((1, D_H)), _full((1, D_H)),
        ],
        out_specs=pl.BlockSpec((_ROWS, D_H), lambda i: (i, 0)),
        out_shape=jax.ShapeDtypeStruct((N, D_H), jnp.float32),
    )(agg, x, wlT, wrT, bl, g, be)


def _dense2(agg, h, wlT, wrT, bl, g, be, wm1T, bm1, wm2, bm2):
    grid = (pl.cdiv(N, _ROWS),)
    return pl.pallas_call(
        _dense2_body,
        grid=grid,
        in_specs=[
            pl.BlockSpec((_ROWS, D_H), lambda i: (i, 0)),
            pl.BlockSpec((_ROWS, D_H), lambda i: (i, 0)),
            _full((D_H, D_H)), _full((D_H, D_H)),
            _full((1, D_H)), _full((1, D_H)), _full((1, D_H)),
            _full((D_H, D_H // 2)), _full((1, D_H // 2)),
            _full((1, D_H // 2)), _full((1, 1)),
        ],
        out_specs=pl.BlockSpec((_ROWS, 1), lambda i: (i, 0)),
        out_shape=jax.ShapeDtypeStruct((N, 1), jnp.float32),
    )(agg, h, wlT, wrT, bl, g, be, wm1T, bm1, wm2, bm2)


def _segmax(feat, src, dst):
    msgs = jnp.take(feat, src, axis=0)
    agg = jax.ops.segment_max(msgs, dst, num_segments=N)
    return jnp.where(jnp.isneginf(agg), 0.0, agg)


def kernel(x, edge_index, W_l1, b_l1, W_r1, W_l2, b_l2, W_r2,
           g1, be1, g2, be2, Wm1, bm1, Wm2, bm2):
    src = edge_index[0].astype(jnp.int32)
    dst = edge_index[1].astype(jnp.int32)
    agg1 = _segmax(x, src, dst)
    h1 = _dense1(agg1, x, W_l1.T, W_r1.T, b_l1[None, :], g1[None, :], be1[None, :])
    agg2 = _segmax(h1, src, dst)
    out = _dense2(agg2, h1, W_l2.T, W_r2.T, b_l2[None, :], g2[None, :], be2[None, :],
                  Wm1.T, bm1[None, :], Wm2[:1, :], bm2[None, :])
    return out[:, 0]


# speedup vs baseline: 1.1297x; 1.1045x over previous
import functools

import jax
import jax.numpy as jnp
from jax import lax
from jax.experimental import pallas as pl
from jax.experimental.pallas import tpu as pltpu
from jax.experimental.pallas import tpu_sc as plsc

N = 10000
E = 320000
D_IN = 128
D_H = 64
EPS = 1e-5


_NC, _NS, _L = 2, 16, 16
_NW = _NC * _NS
_R = 320
_LAST = N - (_NW - 1) * _R
_CH = 2000
_NCH = E // _CH
_SB = 32
_CAP = _CH + _L

_NEG = float("-inf")


def _segmax_body(D, feat_hbm, src_hbm, dst_hbm, out_hbm,
                 srcb, dstb, fsrc, fdst, idxb, stage, acc, sem):
    wid = lax.axis_index("s") * _NC + lax.axis_index("c")
    lo = wid * _R
    hi = lo + _R
    iota = lax.broadcasted_iota(jnp.int32, (_L,), 0)
    nf = D // _L

    def _init_acc(r, _):
        for f in range(nf):
            acc[r, pl.ds(f * _L, _L)] = jnp.full((_L,), _NEG, jnp.float32)
        return 0
    lax.fori_loop(0, _R, _init_acc, 0)

    def _init_fsrc(j, _):
        fsrc[pl.ds(j * _L, _L)] = jnp.zeros((_L,), jnp.int32)
        return 0
    lax.fori_loop(0, _CAP // _L, _init_fsrc, 0)

    def _chunk(c, _):
        pltpu.sync_copy(src_hbm.at[pl.ds(c * _CH, _CH)], srcb)
        pltpu.sync_copy(dst_hbm.at[pl.ds(c * _CH, _CH)], dstb)

        def _scan(j, ptr):
            s = srcb[pl.ds(j * _L, _L)]
            d = dstb[pl.ds(j * _L, _L)]
            m = (d >= lo) & (d < hi)
            mi = jnp.where(m, 1, 0)
            cs = plsc.cumsum(mi)
            pos = ptr + cs - mi
            plsc.store_scatter(fdst, [pos], d - lo, mask=m)
            plsc.store_scatter(fsrc, [pos], s, mask=m)
            return ptr + jnp.max(cs)
        cnt = lax.fori_loop(0, _CH // _L, _scan, jnp.int32(0))

        nb = (cnt + _SB - 1) // _SB

        def _batch(b, _):
            idxb[pl.ds(0, _L)] = fsrc[pl.ds(b * _SB, _L)]
            idxb[pl.ds(_L, _L)] = fsrc[pl.ds(b * _SB + _L, _L)]
            pltpu.async_copy(feat_hbm.at[idxb], stage, sem).wait()
            rmax = jnp.minimum(cnt - b * _SB, _SB)

            def _row(r, _):
                g = b * _SB + r
                dv = fdst[pl.ds(g & -_L, _L)]
                lane = g & (_L - 1)
                dstl = jnp.max(jnp.where(iota == lane, dv, 0))
                for f in range(nf):
                    a = acc[dstl, pl.ds(f * _L, _L)]
                    v = stage[r, pl.ds(f * _L, _L)]
                    acc[dstl, pl.ds(f * _L, _L)] = jnp.maximum(a, v)
                return 0
            lax.fori_loop(0, rmax, _row, 0)
            return 0
        lax.fori_loop(0, nb, _batch, 0)
        return 0
    lax.fori_loop(0, _NCH, _chunk, 0)

    def _fix(r, _):
        for f in range(nf):
            v = acc[r, pl.ds(f * _L, _L)]
            acc[r, pl.ds(f * _L, _L)] = jnp.where(v == _NEG, 0.0, v)
        return 0
    lax.fori_loop(0, _R, _fix, 0)

    @pl.when(wid < _NW - 1)
    def _():
        pltpu.sync_copy(acc.at[pl.ds(0, _R)], out_hbm.at[pl.ds(lo, _R)])

    @pl.when(wid == _NW - 1)
    def _():
        pltpu.sync_copy(acc.at[pl.ds(0, _LAST)], out_hbm.at[pl.ds(lo, _LAST)])


def _sc_segmax(feat, src, dst, D):
    mesh = plsc.VectorSubcoreMesh(core_axis_name="c", subcore_axis_name="s",
                                  num_cores=_NC, num_subcores=_NS)
    return pl.kernel(
        functools.partial(_segmax_body, D),
        out_type=jax.ShapeDtypeStruct((N, D), jnp.float32),
        mesh=mesh,
        compiler_params=pltpu.CompilerParams(needs_layout_passes=False),
        scratch_types=[
            pltpu.VMEM((_CH,), jnp.int32),
            pltpu.VMEM((_CH,), jnp.int32),
            pltpu.VMEM((_CAP,), jnp.int32),
            pltpu.VMEM((_CAP,), jnp.int32),
            pltpu.VMEM((_SB,), jnp.int32),
            pltpu.VMEM((_SB, D_IN), jnp.float32),
            pltpu.VMEM((_R, D), jnp.float32),
            pltpu.SemaphoreType.DMA,
        ],
    )(feat, src, dst)

_ROWS = 1024


def _dense1_body(agg_ref, x_ref, wl_ref, wr_ref, bl_ref, g_ref, be_ref, o_ref):
    agg = agg_ref[...]
    x = x_ref[...]
    h = (jnp.dot(agg, wl_ref[...], preferred_element_type=jnp.float32)
         + jnp.dot(x, wr_ref[...], preferred_element_type=jnp.float32)
         + bl_ref[...])
    mu = jnp.mean(h, axis=-1, keepdims=True)
    var = jnp.mean((h - mu) ** 2, axis=-1, keepdims=True)
    h = (h - mu) * jax.lax.rsqrt(var + EPS) * g_ref[...] + be_ref[...]
    h = jnp.maximum(h, 0.0)
    o_ref[...] = jnp.concatenate([h, h], axis=1)


def _dense2_body(agg_ref, h_ref, wl_ref, wr_ref, bl_ref, g_ref, be_ref,
                 wm1_ref, bm1_ref, wm2_ref, bm2_ref, o_ref):
    agg = agg_ref[...]
    hp = h_ref[:, :D_H]
    h = (jnp.dot(agg, wl_ref[...], preferred_element_type=jnp.float32)
         + jnp.dot(hp, wr_ref[...], preferred_element_type=jnp.float32)
         + bl_ref[...])
    mu = jnp.mean(h, axis=-1, keepdims=True)
    var = jnp.mean((h - mu) ** 2, axis=-1, keepdims=True)
    h = (h - mu) * jax.lax.rsqrt(var + EPS) * g_ref[...] + be_ref[...]
    h = jnp.maximum(h, 0.0)
    m = jnp.maximum(jnp.dot(h, wm1_ref[...], preferred_element_type=jnp.float32)
                    + bm1_ref[...], 0.0)
    z = jnp.sum(m * wm2_ref[...], axis=-1, keepdims=True) + bm2_ref[...]
    o_ref[...] = jax.nn.sigmoid(z)


def _full(shape):
    return pl.BlockSpec(shape, lambda i: tuple(0 for _ in shape))


def _dense1(agg, x, wlT, wrT, bl, g, be):
    grid = (pl.cdiv(N, _ROWS),)
    return pl.pallas_call(
        _dense1_body,
        grid=grid,
        in_specs=[
            pl.BlockSpec((_ROWS, D_IN), lambda i: (i, 0)),
            pl.BlockSpec((_ROWS, D_IN), lambda i: (i, 0)),
            _full((D_IN, D_H)), _full((D_IN, D_H)),
            _full((1, D_H)), _full((1, D_H)), _full((1, D_H)),
        ],
        out_specs=pl.BlockSpec((_ROWS, 2 * D_H), lambda i: (i, 0)),
        out_shape=jax.ShapeDtypeStruct((N, 2 * D_H), jnp.float32),
    )(agg, x, wlT, wrT, bl, g, be)


def _dense2(agg, h, wlT, wrT, bl, g, be, wm1T, bm1, wm2, bm2):
    grid = (pl.cdiv(N, _ROWS),)
    return pl.pallas_call(
        _dense2_body,
        grid=grid,
        in_specs=[
            pl.BlockSpec((_ROWS, D_H), lambda i: (i, 0)),
            pl.BlockSpec((_ROWS, 2 * D_H), lambda i: (i, 0)),
            _full((D_H, D_H)), _full((D_H, D_H)),
            _full((1, D_H)), _full((1, D_H)), _full((1, D_H)),
            _full((D_H, D_H // 2)), _full((1, D_H // 2)),
            _full((1, D_H // 2)), _full((1, 1)),
        ],
        out_specs=pl.BlockSpec((_ROWS, 1), lambda i: (i, 0)),
        out_shape=jax.ShapeDtypeStruct((N, 1), jnp.float32),
    )(agg, h, wlT, wrT, bl, g, be, wm1T, bm1, wm2, bm2)


def kernel(x, edge_index, W_l1, b_l1, W_r1, W_l2, b_l2, W_r2,
           g1, be1, g2, be2, Wm1, bm1, Wm2, bm2):
    src = edge_index[0].astype(jnp.int32)
    dst = edge_index[1].astype(jnp.int32)
    agg1 = _sc_segmax(x, src, dst, D_IN)
    h1 = _dense1(agg1, x, W_l1.T, W_r1.T, b_l1[None, :], g1[None, :], be1[None, :])
    agg2 = _sc_segmax(h1, src, dst, D_H)
    out = _dense2(agg2, h1, W_l2.T, W_r2.T, b_l2[None, :], g2[None, :], be2[None, :],
                  Wm1.T, bm1[None, :], Wm2[:1, :], bm2[None, :])
    return out[:, 0]


# speedup vs baseline: 1.2588x; 1.1143x over previous
import functools

import jax
import jax.numpy as jnp
from jax import lax
from jax.experimental import pallas as pl
from jax.experimental.pallas import tpu as pltpu
from jax.experimental.pallas import tpu_sc as plsc

N = 10000
E = 320000
D_IN = 128
D_H = 64
EPS = 1e-5


_NC, _NS, _L = 2, 16, 16
_NW = _NC * _NS
_R = 320
_LAST = N - (_NW - 1) * _R
_CH = 2000
_NCH = E // _CH
_GB = 128
_FLUSH = 15 * _GB
_CAP = _FLUSH - 1 + _CH + _GB + _L
_TRASH = _R

_NEG = float("-inf")


def _segmax_body(D, feat_hbm, src_hbm, dst_hbm, out_hbm,
                 srcb, dstb, fsrc, fdst, idxb, stage, acc, sem):
    wid = lax.axis_index("s") * _NC + lax.axis_index("c")
    lo = wid * _R
    hi = lo + _R
    iota = lax.broadcasted_iota(jnp.int32, (_L,), 0)
    nf = D // _L
    cols = [iota + f * _L for f in range(nf)]

    def _init_acc(r, _):
        for f in range(nf):
            acc[r, pl.ds(f * _L, _L)] = jnp.full((_L,), _NEG, jnp.float32)
        return 0
    lax.fori_loop(0, _R + 1, _init_acc, 0)

    def _init_fsrc(j, _):
        fsrc[pl.ds(j * _L, _L)] = jnp.zeros((_L,), jnp.int32)
        return 0
    lax.fori_loop(0, _CAP // _L, _init_fsrc, 0)

    def _drain(ptr):
        for k in range(_GB // _L):
            pos = ptr + k * _L + iota
            plsc.store_scatter(fsrc, [pos], jnp.zeros((_L,), jnp.int32))
            plsc.store_scatter(fdst, [pos], jnp.full((_L,), _TRASH, jnp.int32))
        nb = (ptr + _GB - 1) // _GB

        def _batch(b, _):
            for k in range(_GB // _L):
                idxb[pl.ds(k * _L, _L)] = fsrc[pl.ds(b * _GB + k * _L, _L)]
            pltpu.async_copy(feat_hbm.at[idxb], stage, sem).wait()

            def _rowgrp(j, _):
                base = b * _GB + j * _L
                for k in range(_L):
                    rb = plsc.load_gather(fdst, [jnp.full((_L,), base + k, jnp.int32)])
                    r = j * _L + k
                    for f in range(nf):
                        a = plsc.load_gather(acc, [rb, cols[f]])
                        v = stage[r, pl.ds(f * _L, _L)]
                        plsc.store_scatter(acc, [rb, cols[f]], jnp.maximum(a, v))
                return 0
            lax.fori_loop(0, _GB // _L, _rowgrp, 0)
            return 0
        lax.fori_loop(0, nb, _batch, 0)

    def _chunk(c, ptr):
        pltpu.sync_copy(src_hbm.at[pl.ds(c * _CH, _CH)], srcb)
        pltpu.sync_copy(dst_hbm.at[pl.ds(c * _CH, _CH)], dstb)

        def _scan(j, p):
            s = srcb[pl.ds(j * _L, _L)]
            d = dstb[pl.ds(j * _L, _L)]
            m = (d >= lo) & (d < hi)
            mi = jnp.where(m, 1, 0)
            cs = plsc.cumsum(mi)
            pos = p + cs - mi
            plsc.store_scatter(fdst, [pos], d - lo, mask=m)
            plsc.store_scatter(fsrc, [pos], s, mask=m)
            return p + jnp.max(cs)
        ptr = lax.fori_loop(0, _CH // _L, _scan, ptr)

        @pl.when(ptr >= _FLUSH)
        def _():
            _drain(ptr)
        return jnp.where(ptr >= _FLUSH, 0, ptr)
    ptr = lax.fori_loop(0, _NCH, _chunk, jnp.int32(0))

    @pl.when(ptr > 0)
    def _():
        _drain(ptr)

    def _fix(r, _):
        for f in range(nf):
            v = acc[r, pl.ds(f * _L, _L)]
            acc[r, pl.ds(f * _L, _L)] = jnp.where(v == _NEG, 0.0, v)
        return 0
    lax.fori_loop(0, _R, _fix, 0)

    @pl.when(wid < _NW - 1)
    def _():
        pltpu.sync_copy(acc.at[pl.ds(0, _R)], out_hbm.at[pl.ds(lo, _R)])

    @pl.when(wid == _NW - 1)
    def _():
        pltpu.sync_copy(acc.at[pl.ds(0, _LAST)], out_hbm.at[pl.ds(lo, _LAST)])


def _sc_segmax(feat, src, dst, D):
    mesh = plsc.VectorSubcoreMesh(core_axis_name="c", subcore_axis_name="s",
                                  num_cores=_NC, num_subcores=_NS)
    return pl.kernel(
        functools.partial(_segmax_body, D),
        out_type=jax.ShapeDtypeStruct((N, D), jnp.float32),
        mesh=mesh,
        compiler_params=pltpu.CompilerParams(needs_layout_passes=False),
        scratch_types=[
            pltpu.VMEM((_CH,), jnp.int32),
            pltpu.VMEM((_CH,), jnp.int32),
            pltpu.VMEM((_CAP,), jnp.int32),
            pltpu.VMEM((_CAP,), jnp.int32),
            pltpu.VMEM((_GB,), jnp.int32),
            pltpu.VMEM((_GB, D_IN), jnp.float32),
            pltpu.VMEM((_R + 1, D), jnp.float32),
            pltpu.SemaphoreType.DMA,
        ],
    )(feat, src, dst)

_ROWS = 1024


def _dense1_body(agg_ref, x_ref, wl_ref, wr_ref, bl_ref, g_ref, be_ref, o_ref):
    agg = agg_ref[...]
    x = x_ref[...]
    h = (jnp.dot(agg, wl_ref[...], preferred_element_type=jnp.float32)
         + jnp.dot(x, wr_ref[...], preferred_element_type=jnp.float32)
         + bl_ref[...])
    mu = jnp.mean(h, axis=-1, keepdims=True)
    var = jnp.mean((h - mu) ** 2, axis=-1, keepdims=True)
    h = (h - mu) * jax.lax.rsqrt(var + EPS) * g_ref[...] + be_ref[...]
    h = jnp.maximum(h, 0.0)
    o_ref[...] = jnp.concatenate([h, h], axis=1)


def _dense2_body(agg_ref, h_ref, wl_ref, wr_ref, bl_ref, g_ref, be_ref,
                 wm1_ref, bm1_ref, wm2_ref, bm2_ref, o_ref):
    agg = agg_ref[...]
    hp = h_ref[:, :D_H]
    h = (jnp.dot(agg, wl_ref[...], preferred_element_type=jnp.float32)
         + jnp.dot(hp, wr_ref[...], preferred_element_type=jnp.float32)
         + bl_ref[...])
    mu = jnp.mean(h, axis=-1, keepdims=True)
    var = jnp.mean((h - mu) ** 2, axis=-1, keepdims=True)
    h = (h - mu) * jax.lax.rsqrt(var + EPS) * g_ref[...] + be_ref[...]
    h = jnp.maximum(h, 0.0)
    m = jnp.maximum(jnp.dot(h, wm1_ref[...], preferred_element_type=jnp.float32)
                    + bm1_ref[...], 0.0)
    z = jnp.sum(m * wm2_ref[...], axis=-1, keepdims=True) + bm2_ref[...]
    o_ref[...] = jax.nn.sigmoid(z)


def _full(shape):
    return pl.BlockSpec(shape, lambda i: tuple(0 for _ in shape))


def _dense1(agg, x, wlT, wrT, bl, g, be):
    grid = (pl.cdiv(N, _ROWS),)
    return pl.pallas_call(
        _dense1_body,
        grid=grid,
        in_specs=[
            pl.BlockSpec((_ROWS, D_IN), lambda i: (i, 0)),
            pl.BlockSpec((_ROWS, D_IN), lambda i: (i, 0)),
            _full((D_IN, D_H)), _full((D_IN, D_H)),
            _full((1, D_H)), _full((1, D_H)), _full((1, D_H)),
        ],
        out_specs=pl.BlockSpec((_ROWS, 2 * D_H), lambda i: (i, 0)),
        out_shape=jax.ShapeDtypeStruct((N, 2 * D_H), jnp.float32),
    )(agg, x, wlT, wrT, bl, g, be)


def _dense2(agg, h, wlT, wrT, bl, g, be, wm1T, bm1, wm2, bm2):
    grid = (pl.cdiv(N, _ROWS),)
    return pl.pallas_call(
        _dense2_body,
        grid=grid,
        in_specs=[
            pl.BlockSpec((_ROWS, D_H), lambda i: (i, 0)),
            pl.BlockSpec((_ROWS, 2 * D_H), lambda i: (i, 0)),
            _full((D_H, D_H)), _full((D_H, D_H)),
            _full((1, D_H)), _full((1, D_H)), _full((1, D_H)),
            _full((D_H, D_H // 2)), _full((1, D_H // 2)),
            _full((1, D_H // 2)), _full((1, 1)),
        ],
        out_specs=pl.BlockSpec((_ROWS, 1), lambda i: (i, 0)),
        out_shape=jax.ShapeDtypeStruct((N, 1), jnp.float32),
    )(agg, h, wlT, wrT, bl, g, be, wm1T, bm1, wm2, bm2)


def kernel(x, edge_index, W_l1, b_l1, W_r1, W_l2, b_l2, W_r2,
           g1, be1, g2, be2, Wm1, bm1, Wm2, bm2):
    src = edge_index[0].astype(jnp.int32)
    dst = edge_index[1].astype(jnp.int32)
    agg1 = _sc_segmax(x, src, dst, D_IN)
    h1 = _dense1(agg1, x, W_l1.T, W_r1.T, b_l1[None, :], g1[None, :], be1[None, :])
    agg2 = _sc_segmax(h1, src, dst, D_H)
    out = _dense2(agg2, h1, W_l2.T, W_r2.T, b_l2[None, :], g2[None, :], be2[None, :],
                  Wm1.T, bm1[None, :], Wm2[:1, :], bm2[None, :])
    return out[:, 0]


# speedup vs baseline: 1.6957x; 1.3470x over previous
import functools

import jax
import jax.numpy as jnp
from jax import lax
from jax.experimental import pallas as pl
from jax.experimental.pallas import tpu as pltpu
from jax.experimental.pallas import tpu_sc as plsc

N = 10000
E = 320000
D_IN = 128
D_H = 64
EPS = 1e-5


_NC, _NS, _L = 2, 16, 16
_NW = _NC * _NS
_R = 320
_LAST = N - (_NW - 1) * _R
_CH = 8000
_NCH = E // _CH
_GB = 128
_FLUSH = 15 * _GB
_CAP = _FLUSH - 1 + _CH + _GB + _L
_TRASH = _R

_NEG = float("-inf")


def _segmax_body(D, feat_hbm, src_hbm, dst_hbm, out_hbm,
                 srcb, dstb, fsrc, fdst, idxb, stage, acc, sem):
    wid = lax.axis_index("s") * _NC + lax.axis_index("c")
    lo = wid * _R
    hi = lo + _R
    iota = lax.broadcasted_iota(jnp.int32, (_L,), 0)
    nf = D // _L
    cols = [iota + f * _L for f in range(nf)]

    def _init_acc(r, _):
        for f in range(nf):
            acc[r, pl.ds(f * _L, _L)] = jnp.full((_L,), _NEG, jnp.float32)
        return 0
    lax.fori_loop(0, _R + 1, _init_acc, 0)

    def _init_fsrc(j, _):
        fsrc[pl.ds(j * _L, _L)] = jnp.zeros((_L,), jnp.int32)
        return 0
    lax.fori_loop(0, _CAP // _L, _init_fsrc, 0)

    def _drain(ptr):
        for k in range(_GB // _L):
            pos = ptr + k * _L + iota
            plsc.store_scatter(fsrc, [pos], jnp.zeros((_L,), jnp.int32))
            plsc.store_scatter(fdst, [pos], jnp.full((_L,), _TRASH, jnp.int32))
        nb = (ptr + _GB - 1) // _GB

        def _batch(b, _):
            for k in range(_GB // _L):
                idxb[pl.ds(k * _L, _L)] = fsrc[pl.ds(b * _GB + k * _L, _L)]
            pltpu.async_copy(feat_hbm.at[idxb], stage, sem).wait()

            def _rowgrp(j, _):
                base = b * _GB + j * _L
                for k in range(_L):
                    rb = plsc.load_gather(fdst, [jnp.full((_L,), base + k, jnp.int32)])
                    r = j * _L + k
                    for f in range(nf):
                        a = plsc.load_gather(acc, [rb, cols[f]])
                        v = stage[r, pl.ds(f * _L, _L)]
                        plsc.store_scatter(acc, [rb, cols[f]], jnp.maximum(a, v))
                return 0
            lax.fori_loop(0, _GB // _L, _rowgrp, 0)
            return 0
        lax.fori_loop(0, nb, _batch, 0)

    def _chunk(c, ptr_v):
        pltpu.sync_copy(src_hbm.at[pl.ds(c * _CH, _CH)], srcb)
        pltpu.sync_copy(dst_hbm.at[pl.ds(c * _CH, _CH)], dstb)

        def _scan(j, p_v):
            s = srcb[pl.ds(j * _L, _L)]
            d = dstb[pl.ds(j * _L, _L)]
            m = (d >= lo) & (d < hi)
            mi = jnp.where(m, 1, 0)
            cs = plsc.cumsum(mi)
            pos = p_v + cs - mi
            plsc.store_scatter(fdst, [pos], d - lo, mask=m)
            plsc.store_scatter(fsrc, [pos], s, mask=m)
            return p_v + plsc.all_reduce_population_count(m)
        ptr_v = lax.fori_loop(0, _CH // _L, _scan, ptr_v)
        ptr = jnp.max(ptr_v)

        @pl.when(ptr >= _FLUSH)
        def _():
            _drain(ptr)
        return jnp.where(ptr >= _FLUSH, 0, ptr_v)
    ptr_v = lax.fori_loop(0, _NCH, _chunk, jnp.zeros((_L,), jnp.int32))
    ptr = jnp.max(ptr_v)

    @pl.when(ptr > 0)
    def _():
        _drain(ptr)

    def _fix(r, _):
        for f in range(nf):
            v = acc[r, pl.ds(f * _L, _L)]
            acc[r, pl.ds(f * _L, _L)] = jnp.where(v == _NEG, 0.0, v)
        return 0
    lax.fori_loop(0, _R, _fix, 0)

    @pl.when(wid < _NW - 1)
    def _():
        pltpu.sync_copy(acc.at[pl.ds(0, _R)], out_hbm.at[pl.ds(lo, _R)])

    @pl.when(wid == _NW - 1)
    def _():
        pltpu.sync_copy(acc.at[pl.ds(0, _LAST)], out_hbm.at[pl.ds(lo, _LAST)])


def _sc_segmax(feat, src, dst, D):
    mesh = plsc.VectorSubcoreMesh(core_axis_name="c", subcore_axis_name="s",
                                  num_cores=_NC, num_subcores=_NS)
    return pl.kernel(
        functools.partial(_segmax_body, D),
        out_type=jax.ShapeDtypeStruct((N, D), jnp.float32),
        mesh=mesh,
        compiler_params=pltpu.CompilerParams(needs_layout_passes=False),
        scratch_types=[
            pltpu.VMEM((_CH,), jnp.int32),
            pltpu.VMEM((_CH,), jnp.int32),
            pltpu.VMEM((_CAP,), jnp.int32),
            pltpu.VMEM((_CAP,), jnp.int32),
            pltpu.VMEM((_GB,), jnp.int32),
            pltpu.VMEM((_GB, D_IN), jnp.float32),
            pltpu.VMEM((_R + 1, D), jnp.float32),
            pltpu.SemaphoreType.DMA,
        ],
    )(feat, src, dst)

_ROWS = 1024


def _dense1_body(agg_ref, x_ref, wl_ref, wr_ref, bl_ref, g_ref, be_ref, o_ref):
    agg = agg_ref[...]
    x = x_ref[...]
    h = (jnp.dot(agg, wl_ref[...], preferred_element_type=jnp.float32)
         + jnp.dot(x, wr_ref[...], preferred_element_type=jnp.float32)
         + bl_ref[...])
    mu = jnp.mean(h, axis=-1, keepdims=True)
    var = jnp.mean((h - mu) ** 2, axis=-1, keepdims=True)
    h = (h - mu) * jax.lax.rsqrt(var + EPS) * g_ref[...] + be_ref[...]
    h = jnp.maximum(h, 0.0)
    o_ref[...] = jnp.concatenate([h, h], axis=1)


def _dense2_body(agg_ref, h_ref, wl_ref, wr_ref, bl_ref, g_ref, be_ref,
                 wm1_ref, bm1_ref, wm2_ref, bm2_ref, o_ref):
    agg = agg_ref[...]
    hp = h_ref[:, :D_H]
    h = (jnp.dot(agg, wl_ref[...], preferred_element_type=jnp.float32)
         + jnp.dot(hp, wr_ref[...], preferred_element_type=jnp.float32)
         + bl_ref[...])
    mu = jnp.mean(h, axis=-1, keepdims=True)
    var = jnp.mean((h - mu) ** 2, axis=-1, keepdims=True)
    h = (h - mu) * jax.lax.rsqrt(var + EPS) * g_ref[...] + be_ref[...]
    h = jnp.maximum(h, 0.0)
    m = jnp.maximum(jnp.dot(h, wm1_ref[...], preferred_element_type=jnp.float32)
                    + bm1_ref[...], 0.0)
    z = jnp.sum(m * wm2_ref[...], axis=-1, keepdims=True) + bm2_ref[...]
    o_ref[...] = jax.nn.sigmoid(z)


def _full(shape):
    return pl.BlockSpec(shape, lambda i: tuple(0 for _ in shape))


def _dense1(agg, x, wlT, wrT, bl, g, be):
    grid = (pl.cdiv(N, _ROWS),)
    return pl.pallas_call(
        _dense1_body,
        grid=grid,
        in_specs=[
            pl.BlockSpec((_ROWS, D_IN), lambda i: (i, 0)),
            pl.BlockSpec((_ROWS, D_IN), lambda i: (i, 0)),
            _full((D_IN, D_H)), _full((D_IN, D_H)),
            _full((1, D_H)), _full((1, D_H)), _full((1, D_H)),
        ],
        out_specs=pl.BlockSpec((_ROWS, 2 * D_H), lambda i: (i, 0)),
        out_shape=jax.ShapeDtypeStruct((N, 2 * D_H), jnp.float32),
    )(agg, x, wlT, wrT, bl, g, be)


def _dense2(agg, h, wlT, wrT, bl, g, be, wm1T, bm1, wm2, bm2):
    grid = (pl.cdiv(N, _ROWS),)
    return pl.pallas_call(
        _dense2_body,
        grid=grid,
        in_specs=[
            pl.BlockSpec((_ROWS, D_H), lambda i: (i, 0)),
            pl.BlockSpec((_ROWS, 2 * D_H), lambda i: (i, 0)),
            _full((D_H, D_H)), _full((D_H, D_H)),
            _full((1, D_H)), _full((1, D_H)), _full((1, D_H)),
            _full((D_H, D_H // 2)), _full((1, D_H // 2)),
            _full((1, D_H // 2)), _full((1, 1)),
        ],
        out_specs=pl.BlockSpec((_ROWS, 1), lambda i: (i, 0)),
        out_shape=jax.ShapeDtypeStruct((N, 1), jnp.float32),
    )(agg, h, wlT, wrT, bl, g, be, wm1T, bm1, wm2, bm2)


def kernel(x, edge_index, W_l1, b_l1, W_r1, W_l2, b_l2, W_r2,
           g1, be1, g2, be2, Wm1, bm1, Wm2, bm2):
    src = edge_index[0].astype(jnp.int32)
    dst = edge_index[1].astype(jnp.int32)
    agg1 = _sc_segmax(x, src, dst, D_IN)
    h1 = _dense1(agg1, x, W_l1.T, W_r1.T, b_l1[None, :], g1[None, :], be1[None, :])
    agg2 = _sc_segmax(h1, src, dst, D_H)
    out = _dense2(agg2, h1, W_l2.T, W_r2.T, b_l2[None, :], g2[None, :], be2[None, :],
                  Wm1.T, bm1[None, :], Wm2[:1, :], bm2[None, :])
    return out[:, 0]


# speedup vs baseline: 2.0990x; 1.2379x over previous
import functools

import jax
import jax.numpy as jnp
from jax import lax
from jax.experimental import pallas as pl
from jax.experimental.pallas import tpu as pltpu
from jax.experimental.pallas import tpu_sc as plsc

N = 10000
E = 320000
D_IN = 128
D_H = 64
EPS = 1e-5


_NC, _NS, _L = 2, 16, 16
_NW = _NC * _NS
_R = 320
_LAST = N - (_NW - 1) * _R
_CH = 6400
_NCH = E // _CH
_GB = 128
_FLUSH = 15 * _GB
_CAP = _FLUSH - 1 + _CH + _GB + _L
_TRASH = _R

_NEG = float("-inf")


def _segmax_body(D, feat_hbm, src_hbm, dst_hbm, out_hbm,
                 srcb0, dstb0, srcb1, dstb1, fsrc, fdst,
                 idx0, idx1, stage0, stage1, acc,
                 csem0, csem1, gsem0, gsem1):
    wid = lax.axis_index("s") * _NC + lax.axis_index("c")
    lo = wid * _R
    hi = lo + _R
    iota = lax.broadcasted_iota(jnp.int32, (_L,), 0)
    nf = D // _L
    cols = [iota + f * _L for f in range(nf)]

    def _init_acc(r, _):
        for f in range(nf):
            acc[r, pl.ds(f * _L, _L)] = jnp.full((_L,), _NEG, jnp.float32)
        return 0
    lax.fori_loop(0, _R + 1, _init_acc, 0)

    def _init_fsrc(j, _):
        fsrc[pl.ds(j * _L, _L)] = jnp.zeros((_L,), jnp.int32)
        return 0
    lax.fori_loop(0, _CAP // _L, _init_fsrc, 0)

    def _fire_chunk(c, sb, db, csem):
        pltpu.async_copy(src_hbm.at[pl.ds(c * _CH, _CH)], sb, csem)
        pltpu.async_copy(dst_hbm.at[pl.ds(c * _CH, _CH)], db, csem)

    def _wait_chunk(c, sb, db, csem):
        pltpu.make_async_copy(src_hbm.at[pl.ds(c * _CH, _CH)], sb, csem).wait()
        pltpu.make_async_copy(dst_hbm.at[pl.ds(c * _CH, _CH)], db, csem).wait()

    def _fill_idx(idxr, b):
        for k in range(_GB // _L):
            idxr[pl.ds(k * _L, _L)] = fsrc[pl.ds(b * _GB + k * _L, _L)]

    def _fire_gather(idxr, st, gs):
        pltpu.async_copy(feat_hbm.at[idxr], st, gs)

    def _wait_gather(idxr, st, gs):
        pltpu.make_async_copy(feat_hbm.at[idxr], st, gs).wait()

    def _process(st, b):
        def _rowgrp(j, _):
            base = b * _GB + j * _L
            for k in range(_L):
                rb = plsc.load_gather(fdst, [jnp.full((_L,), base + k, jnp.int32)])
                r = j * _L + k
                for f in range(nf):
                    a = plsc.load_gather(acc, [rb, cols[f]])
                    v = st[r, pl.ds(f * _L, _L)]
                    plsc.store_scatter(acc, [rb, cols[f]], jnp.maximum(a, v))
            return 0
        lax.fori_loop(0, _GB // _L, _rowgrp, 0)

    def _drain(ptr):
        for k in range(_GB // _L):
            pos = ptr + k * _L + iota
            plsc.store_scatter(fsrc, [pos], jnp.zeros((_L,), jnp.int32))
            plsc.store_scatter(fdst, [pos], jnp.full((_L,), _TRASH, jnp.int32))
        nb = (ptr + _GB - 1) // _GB
        _fill_idx(idx0, 0)
        _fire_gather(idx0, stage0, gsem0)

        def _bpair(g, _):
            b0 = 2 * g
            b1 = b0 + 1

            @pl.when(b1 < nb)
            def _():
                _fill_idx(idx1, b1)
                _fire_gather(idx1, stage1, gsem1)
            _wait_gather(idx0, stage0, gsem0)
            _process(stage0, b0)

            @pl.when(b1 < nb)
            def _():
                @pl.when(b1 + 1 < nb)
                def _():
                    _fill_idx(idx0, b1 + 1)
                    _fire_gather(idx0, stage0, gsem0)
                _wait_gather(idx1, stage1, gsem1)
                _process(stage1, b1)
            return 0
        lax.fori_loop(0, (nb + 1) // 2, _bpair, 0)

    def _scan_buf(sb, db, ptr_v):
        def _scan(j, p_v):
            s = sb[pl.ds(j * _L, _L)]
            d = db[pl.ds(j * _L, _L)]
            m = (d >= lo) & (d < hi)
            mi = jnp.where(m, 1, 0)
            cs = plsc.cumsum(mi)
            pos = p_v + cs - mi
            plsc.store_scatter(fdst, [pos], d - lo, mask=m)
            plsc.store_scatter(fsrc, [pos], s, mask=m)
            return p_v + plsc.all_reduce_population_count(m)
        return lax.fori_loop(0, _CH // _L, _scan, ptr_v)

    def _scan_drain(ptr_v):
        ptr = jnp.max(ptr_v)

        @pl.when(ptr >= _FLUSH)
        def _():
            _drain(ptr)
        return jnp.where(ptr >= _FLUSH, 0, ptr_v)

    _fire_chunk(0, srcb0, dstb0, csem0)

    def _pair(p, ptr_v):
        c0 = 2 * p
        _fire_chunk(c0 + 1, srcb1, dstb1, csem1)
        _wait_chunk(c0, srcb0, dstb0, csem0)
        ptr_v = _scan_buf(srcb0, dstb0, ptr_v)
        ptr_v = _scan_drain(ptr_v)

        @pl.when(c0 + 2 < _NCH)
        def _():
            _fire_chunk(c0 + 2, srcb0, dstb0, csem0)
        _wait_chunk(c0 + 1, srcb1, dstb1, csem1)
        ptr_v = _scan_buf(srcb1, dstb1, ptr_v)
        ptr_v = _scan_drain(ptr_v)
        return ptr_v
    ptr_v = lax.fori_loop(0, _NCH // 2, _pair, jnp.zeros((_L,), jnp.int32))
    ptr = jnp.max(ptr_v)

    @pl.when(ptr > 0)
    def _():
        _drain(ptr)

    def _fix(r, _):
        for f in range(nf):
            v = acc[r, pl.ds(f * _L, _L)]
            acc[r, pl.ds(f * _L, _L)] = jnp.where(v == _NEG, 0.0, v)
        return 0
    lax.fori_loop(0, _R, _fix, 0)

    @pl.when(wid < _NW - 1)
    def _():
        pltpu.sync_copy(acc.at[pl.ds(0, _R)], out_hbm.at[pl.ds(lo, _R)])

    @pl.when(wid == _NW - 1)
    def _():
        pltpu.sync_copy(acc.at[pl.ds(0, _LAST)], out_hbm.at[pl.ds(lo, _LAST)])


def _sc_segmax(feat, src, dst, D):
    mesh = plsc.VectorSubcoreMesh(core_axis_name="c", subcore_axis_name="s",
                                  num_cores=_NC, num_subcores=_NS)
    return pl.kernel(
        functools.partial(_segmax_body, D),
        out_type=jax.ShapeDtypeStruct((N, D), jnp.float32),
        mesh=mesh,
        compiler_params=pltpu.CompilerParams(needs_layout_passes=False),
        scratch_types=[
            pltpu.VMEM((_CH,), jnp.int32),
            pltpu.VMEM((_CH,), jnp.int32),
            pltpu.VMEM((_CH,), jnp.int32),
            pltpu.VMEM((_CH,), jnp.int32),
            pltpu.VMEM((_CAP,), jnp.int32),
            pltpu.VMEM((_CAP,), jnp.int32),
            pltpu.VMEM((_GB,), jnp.int32),
            pltpu.VMEM((_GB,), jnp.int32),
            pltpu.VMEM((_GB, D_IN), jnp.float32),
            pltpu.VMEM((_GB, D_IN), jnp.float32),
            pltpu.VMEM((_R + 1, D), jnp.float32),
            pltpu.SemaphoreType.DMA,
            pltpu.SemaphoreType.DMA,
            pltpu.SemaphoreType.DMA,
            pltpu.SemaphoreType.DMA,
        ],
    )(feat, src, dst)

_ROWS = 1024


def _dense1_body(agg_ref, x_ref, wl_ref, wr_ref, bl_ref, g_ref, be_ref, o_ref):
    agg = agg_ref[...]
    x = x_ref[...]
    h = (jnp.dot(agg, wl_ref[...], preferred_element_type=jnp.float32)
         + jnp.dot(x, wr_ref[...], preferred_element_type=jnp.float32)
         + bl_ref[...])
    mu = jnp.mean(h, axis=-1, keepdims=True)
    var = jnp.mean((h - mu) ** 2, axis=-1, keepdims=True)
    h = (h - mu) * jax.lax.rsqrt(var + EPS) * g_ref[...] + be_ref[...]
    h = jnp.maximum(h, 0.0)
    o_ref[...] = jnp.concatenate([h, h], axis=1)


def _dense2_body(agg_ref, h_ref, wl_ref, wr_ref, bl_ref, g_ref, be_ref,
                 wm1_ref, bm1_ref, wm2_ref, bm2_ref, o_ref):
    agg = agg_ref[...]
    hp = h_ref[:, :D_H]
    h = (jnp.dot(agg, wl_ref[...], preferred_element_type=jnp.float32)
         + jnp.dot(hp, wr_ref[...], preferred_element_type=jnp.float32)
         + bl_ref[...])
    mu = jnp.mean(h, axis=-1, keepdims=True)
    var = jnp.mean((h - mu) ** 2, axis=-1, keepdims=True)
    h = (h - mu) * jax.lax.rsqrt(var + EPS) * g_ref[...] + be_ref[...]
    h = jnp.maximum(h, 0.0)
    m = jnp.maximum(jnp.dot(h, wm1_ref[...], preferred_element_type=jnp.float32)
                    + bm1_ref[...], 0.0)
    z = jnp.sum(m * wm2_ref[...], axis=-1, keepdims=True) + bm2_ref[...]
    o_ref[...] = jax.nn.sigmoid(z)


def _full(shape):
    return pl.BlockSpec(shape, lambda i: tuple(0 for _ in shape))


def _dense1(agg, x, wlT, wrT, bl, g, be):
    grid = (pl.cdiv(N, _ROWS),)
    return pl.pallas_call(
        _dense1_body,
        grid=grid,
        in_specs=[
            pl.BlockSpec((_ROWS, D_IN), lambda i: (i, 0)),
            pl.BlockSpec((_ROWS, D_IN), lambda i: (i, 0)),
            _full((D_IN, D_H)), _full((D_IN, D_H)),
            _full((1, D_H)), _full((1, D_H)), _full((1, D_H)),
        ],
        out_specs=pl.BlockSpec((_ROWS, 2 * D_H), lambda i: (i, 0)),
        out_shape=jax.ShapeDtypeStruct((N, 2 * D_H), jnp.float32),
    )(agg, x, wlT, wrT, bl, g, be)


def _dense2(agg, h, wlT, wrT, bl, g, be, wm1T, bm1, wm2, bm2):
    grid = (pl.cdiv(N, _ROWS),)
    return pl.pallas_call(
        _dense2_body,
        grid=grid,
        in_specs=[
            pl.BlockSpec((_ROWS, D_H), lambda i: (i, 0)),
            pl.BlockSpec((_ROWS, 2 * D_H), lambda i: (i, 0)),
            _full((D_H, D_H)), _full((D_H, D_H)),
            _full((1, D_H)), _full((1, D_H)), _full((1, D_H)),
            _full((D_H, D_H // 2)), _full((1, D_H // 2)),
            _full((1, D_H // 2)), _full((1, 1)),
        ],
        out_specs=pl.BlockSpec((_ROWS, 1), lambda i: (i, 0)),
        out_shape=jax.ShapeDtypeStruct((N, 1), jnp.float32),
    )(agg, h, wlT, wrT, bl, g, be, wm1T, bm1, wm2, bm2)


def kernel(x, edge_index, W_l1, b_l1, W_r1, W_l2, b_l2, W_r2,
           g1, be1, g2, be2, Wm1, bm1, Wm2, bm2):
    src = edge_index[0].astype(jnp.int32)
    dst = edge_index[1].astype(jnp.int32)
    agg1 = _sc_segmax(x, src, dst, D_IN)
    h1 = _dense1(agg1, x, W_l1.T, W_r1.T, b_l1[None, :], g1[None, :], be1[None, :])
    agg2 = _sc_segmax(h1, src, dst, D_H)
    out = _dense2(agg2, h1, W_l2.T, W_r2.T, b_l2[None, :], g2[None, :], be2[None, :],
                  Wm1.T, bm1[None, :], Wm2[:1, :], bm2[None, :])
    return out[:, 0]


# speedup vs baseline: 2.4158x; 1.1509x over previous
import functools

import jax
import jax.numpy as jnp
from jax import lax
from jax.experimental import pallas as pl
from jax.experimental.pallas import tpu as pltpu
from jax.experimental.pallas import tpu_sc as plsc

N = 10000
E = 320000
D_IN = 128
D_H = 64
EPS = 1e-5


_NC, _NS, _L = 2, 16, 16
_NW = _NC * _NS
_R = 320
_LAST = N - (_NW - 1) * _R
_CH = 6400
_NCH = E // _CH
_GB = 128
_FLUSH = 15 * _GB
_CAP = _FLUSH - 1 + _CH + _GB + _L
_TRASH = _R

_NEG = float("-inf")


def _segmax_body(D, feat_hbm, src_hbm, dst_hbm, out_hbm,
                 srcb0, dstb0, srcb1, dstb1, fsrc, fdst,
                 idx0, idx1, stage0, stage1, acc,
                 csem0, csem1, gsem0, gsem1):
    wid = lax.axis_index("s") * _NC + lax.axis_index("c")
    lo = wid * _R
    hi = lo + _R
    iota = lax.broadcasted_iota(jnp.int32, (_L,), 0)
    nf = D // _L
    cols = [iota + f * _L for f in range(nf)]

    def _init_acc(r, _):
        for f in range(nf):
            acc[r, pl.ds(f * _L, _L)] = jnp.full((_L,), _NEG, jnp.float32)
        return 0
    lax.fori_loop(0, _R + 1, _init_acc, 0)

    def _init_fsrc(j, _):
        fsrc[pl.ds(j * _L, _L)] = jnp.zeros((_L,), jnp.int32)
        return 0
    lax.fori_loop(0, _CAP // _L, _init_fsrc, 0)

    def _fire_chunk(c, sb, db, csem):
        pltpu.async_copy(src_hbm.at[pl.ds(c * _CH, _CH)], sb, csem)
        pltpu.async_copy(dst_hbm.at[pl.ds(c * _CH, _CH)], db, csem)

    def _wait_chunk(c, sb, db, csem):
        pltpu.make_async_copy(src_hbm.at[pl.ds(c * _CH, _CH)], sb, csem).wait()
        pltpu.make_async_copy(dst_hbm.at[pl.ds(c * _CH, _CH)], db, csem).wait()

    def _fill_idx(idxr, b):
        for k in range(_GB // _L):
            idxr[pl.ds(k * _L, _L)] = fsrc[pl.ds(b * _GB + k * _L, _L)]

    def _fire_gather(idxr, st, gs):
        pltpu.async_copy(feat_hbm.at[idxr], st, gs)

    def _wait_gather(idxr, st, gs):
        pltpu.make_async_copy(feat_hbm.at[idxr], st, gs).wait()

    def _process(st, b):
        def _rowgrp(j, _):
            base = b * _GB + j * _L
            for k in range(_L):
                rb = plsc.load_gather(fdst, [jnp.full((_L,), base + k, jnp.int32)])
                r = j * _L + k
                for f in range(nf):
                    a = plsc.load_gather(acc, [rb, cols[f]])
                    v = st[r, pl.ds(f * _L, _L)]
                    plsc.store_scatter(acc, [rb, cols[f]], jnp.maximum(a, v))
            return 0
        lax.fori_loop(0, _GB // _L, _rowgrp, 0)

    def _drain(ptr):
        for k in range(_GB // _L):
            pos = ptr + k * _L + iota
            plsc.store_scatter(fsrc, [pos], jnp.zeros((_L,), jnp.int32))
            plsc.store_scatter(fdst, [pos], jnp.full((_L,), _TRASH, jnp.int32))
        nb = (ptr + _GB - 1) // _GB
        _fill_idx(idx0, 0)
        _fire_gather(idx0, stage0, gsem0)

        def _bpair(g, _):
            b0 = 2 * g
            b1 = b0 + 1

            @pl.when(b1 < nb)
            def _():
                _fill_idx(idx1, b1)
                _fire_gather(idx1, stage1, gsem1)
            _wait_gather(idx0, stage0, gsem0)
            _process(stage0, b0)

            @pl.when(b1 < nb)
            def _():
                @pl.when(b1 + 1 < nb)
                def _():
                    _fill_idx(idx0, b1 + 1)
                    _fire_gather(idx0, stage0, gsem0)
                _wait_gather(idx1, stage1, gsem1)
                _process(stage1, b1)
            return 0
        lax.fori_loop(0, (nb + 1) // 2, _bpair, 0)

    def _scan_buf(sb, db, ptr_v):
        U = 4

        def _scan(j, p_v):
            base = j * (U * _L)
            sv = [sb[pl.ds(base + u * _L, _L)] for u in range(U)]
            dv = [db[pl.ds(base + u * _L, _L)] for u in range(U)]
            ms = [(d >= lo) & (d < hi) for d in dv]
            cnt = [plsc.all_reduce_population_count(m) for m in ms]
            offs = [p_v]
            for u in range(1, U):
                offs.append(offs[-1] + cnt[u - 1])
            for u in range(U):
                mi = jnp.where(ms[u], 1, 0)
                cs = plsc.cumsum(mi)
                pos = offs[u] + cs - mi
                plsc.store_scatter(fdst, [pos], dv[u] - lo, mask=ms[u])
                plsc.store_scatter(fsrc, [pos], sv[u], mask=ms[u])
            return offs[-1] + cnt[-1]
        return lax.fori_loop(0, _CH // (U * _L), _scan, ptr_v)

    def _scan_drain(ptr_v):
        ptr = jnp.max(ptr_v)

        @pl.when(ptr >= _FLUSH)
        def _():
            _drain(ptr)
        return jnp.where(ptr >= _FLUSH, 0, ptr_v)

    _fire_chunk(0, srcb0, dstb0, csem0)

    def _pair(p, ptr_v):
        c0 = 2 * p
        _fire_chunk(c0 + 1, srcb1, dstb1, csem1)
        _wait_chunk(c0, srcb0, dstb0, csem0)
        ptr_v = _scan_buf(srcb0, dstb0, ptr_v)
        ptr_v = _scan_drain(ptr_v)

        @pl.when(c0 + 2 < _NCH)
        def _():
            _fire_chunk(c0 + 2, srcb0, dstb0, csem0)
        _wait_chunk(c0 + 1, srcb1, dstb1, csem1)
        ptr_v = _scan_buf(srcb1, dstb1, ptr_v)
        ptr_v = _scan_drain(ptr_v)
        return ptr_v
    ptr_v = lax.fori_loop(0, _NCH // 2, _pair, jnp.zeros((_L,), jnp.int32))
    ptr = jnp.max(ptr_v)

    @pl.when(ptr > 0)
    def _():
        _drain(ptr)

    def _fix(r, _):
        for f in range(nf):
            v = acc[r, pl.ds(f * _L, _L)]
            acc[r, pl.ds(f * _L, _L)] = jnp.where(v == _NEG, 0.0, v)
        return 0
    lax.fori_loop(0, _R, _fix, 0)

    @pl.when(wid < _NW - 1)
    def _():
        pltpu.sync_copy(acc.at[pl.ds(0, _R)], out_hbm.at[pl.ds(lo, _R)])

    @pl.when(wid == _NW - 1)
    def _():
        pltpu.sync_copy(acc.at[pl.ds(0, _LAST)], out_hbm.at[pl.ds(lo, _LAST)])


def _sc_segmax(feat, src, dst, D):
    mesh = plsc.VectorSubcoreMesh(core_axis_name="c", subcore_axis_name="s",
                                  num_cores=_NC, num_subcores=_NS)
    return pl.kernel(
        functools.partial(_segmax_body, D),
        out_type=jax.ShapeDtypeStruct((N, D), jnp.float32),
        mesh=mesh,
        compiler_params=pltpu.CompilerParams(needs_layout_passes=False),
        scratch_types=[
            pltpu.VMEM((_CH,), jnp.int32),
            pltpu.VMEM((_CH,), jnp.int32),
            pltpu.VMEM((_CH,), jnp.int32),
            pltpu.VMEM((_CH,), jnp.int32),
            pltpu.VMEM((_CAP,), jnp.int32),
            pltpu.VMEM((_CAP,), jnp.int32),
            pltpu.VMEM((_GB,), jnp.int32),
            pltpu.VMEM((_GB,), jnp.int32),
            pltpu.VMEM((_GB, D_IN), jnp.float32),
            pltpu.VMEM((_GB, D_IN), jnp.float32),
            pltpu.VMEM((_R + 1, D), jnp.float32),
            pltpu.SemaphoreType.DMA,
            pltpu.SemaphoreType.DMA,
            pltpu.SemaphoreType.DMA,
            pltpu.SemaphoreType.DMA,
        ],
    )(feat, src, dst)

_ROWS = 1024


def _dense1_body(agg_ref, x_ref, wl_ref, wr_ref, bl_ref, g_ref, be_ref, o_ref):
    agg = agg_ref[...]
    x = x_ref[...]
    h = (jnp.dot(agg, wl_ref[...], preferred_element_type=jnp.float32)
         + jnp.dot(x, wr_ref[...], preferred_element_type=jnp.float32)
         + bl_ref[...])
    mu = jnp.mean(h, axis=-1, keepdims=True)
    var = jnp.mean((h - mu) ** 2, axis=-1, keepdims=True)
    h = (h - mu) * jax.lax.rsqrt(var + EPS) * g_ref[...] + be_ref[...]
    h = jnp.maximum(h, 0.0)
    o_ref[...] = jnp.concatenate([h, h], axis=1)


def _dense2_body(agg_ref, h_ref, wl_ref, wr_ref, bl_ref, g_ref, be_ref,
                 wm1_ref, bm1_ref, wm2_ref, bm2_ref, o_ref):
    agg = agg_ref[...]
    hp = h_ref[:, :D_H]
    h = (jnp.dot(agg, wl_ref[...], preferred_element_type=jnp.float32)
         + jnp.dot(hp, wr_ref[...], preferred_element_type=jnp.float32)
         + bl_ref[...])
    mu = jnp.mean(h, axis=-1, keepdims=True)
    var = jnp.mean((h - mu) ** 2, axis=-1, keepdims=True)
    h = (h - mu) * jax.lax.rsqrt(var + EPS) * g_ref[...] + be_ref[...]
    h = jnp.maximum(h, 0.0)
    m = jnp.maximum(jnp.dot(h, wm1_ref[...], preferred_element_type=jnp.float32)
                    + bm1_ref[...], 0.0)
    z = jnp.sum(m * wm2_ref[...], axis=-1, keepdims=True) + bm2_ref[...]
    o_ref[...] = jax.nn.sigmoid(z)


def _full(shape):
    return pl.BlockSpec(shape, lambda i: tuple(0 for _ in shape))


def _dense1(agg, x, wlT, wrT, bl, g, be):
    grid = (pl.cdiv(N, _ROWS),)
    return pl.pallas_call(
        _dense1_body,
        grid=grid,
        in_specs=[
            pl.BlockSpec((_ROWS, D_IN), lambda i: (i, 0)),
            pl.BlockSpec((_ROWS, D_IN), lambda i: (i, 0)),
            _full((D_IN, D_H)), _full((D_IN, D_H)),
            _full((1, D_H)), _full((1, D_H)), _full((1, D_H)),
        ],
        out_specs=pl.BlockSpec((_ROWS, 2 * D_H), lambda i: (i, 0)),
        out_shape=jax.ShapeDtypeStruct((N, 2 * D_H), jnp.float32),
    )(agg, x, wlT, wrT, bl, g, be)


def _dense2(agg, h, wlT, wrT, bl, g, be, wm1T, bm1, wm2, bm2):
    grid = (pl.cdiv(N, _ROWS),)
    return pl.pallas_call(
        _dense2_body,
        grid=grid,
        in_specs=[
            pl.BlockSpec((_ROWS, D_H), lambda i: (i, 0)),
            pl.BlockSpec((_ROWS, 2 * D_H), lambda i: (i, 0)),
            _full((D_H, D_H)), _full((D_H, D_H)),
            _full((1, D_H)), _full((1, D_H)), _full((1, D_H)),
            _full((D_H, D_H // 2)), _full((1, D_H // 2)),
            _full((1, D_H // 2)), _full((1, 1)),
        ],
        out_specs=pl.BlockSpec((_ROWS, 1), lambda i: (i, 0)),
        out_shape=jax.ShapeDtypeStruct((N, 1), jnp.float32),
    )(agg, h, wlT, wrT, bl, g, be, wm1T, bm1, wm2, bm2)


def kernel(x, edge_index, W_l1, b_l1, W_r1, W_l2, b_l2, W_r2,
           g1, be1, g2, be2, Wm1, bm1, Wm2, bm2):
    src = edge_index[0].astype(jnp.int32)
    dst = edge_index[1].astype(jnp.int32)
    agg1 = _sc_segmax(x, src, dst, D_IN)
    h1 = _dense1(agg1, x, W_l1.T, W_r1.T, b_l1[None, :], g1[None, :], be1[None, :])
    agg2 = _sc_segmax(h1, src, dst, D_H)
    out = _dense2(agg2, h1, W_l2.T, W_r2.T, b_l2[None, :], g2[None, :], be2[None, :],
                  Wm1.T, bm1[None, :], Wm2[:1, :], bm2[None, :])
    return out[:, 0]


# speedup vs baseline: 2.4879x; 1.0298x over previous
import functools

import jax
import jax.numpy as jnp
from jax import lax
from jax.experimental import pallas as pl
from jax.experimental.pallas import tpu as pltpu
from jax.experimental.pallas import tpu_sc as plsc

N = 10000
E = 320000
D_IN = 128
D_H = 64
EPS = 1e-5


_NC, _NS, _L = 2, 16, 16
_NW = _NC * _NS
_R = 320
_LAST = N - (_NW - 1) * _R
_CH = 6400
_NCH = E // _CH
_GB = 128
_FLUSH = 15 * _GB
_CAP = _FLUSH - 1 + _CH + _GB + _L
_TRASH = _R

_NEG = float("-inf")


def _segmax_body(D, feat_hbm, src_hbm, dst_hbm, out_hbm,
                 srcb0, dstb0, srcb1, dstb1, fsrc, fdst,
                 idx0, idx1, stage0, stage1, acc,
                 csem0, csem1, gsem0, gsem1):
    wid = lax.axis_index("s") * _NC + lax.axis_index("c")
    lo = wid * _R
    hi = lo + _R
    iota = lax.broadcasted_iota(jnp.int32, (_L,), 0)
    nw = D // 32
    cols = [iota + f * _L for f in range(nw)]
    neg_pair = plsc.bitcast(jnp.full((2 * _L,), _NEG, jnp.bfloat16), jnp.int32)

    def _init_acc(r, _):
        for f in range(nw):
            acc[r, pl.ds(f * _L, _L)] = neg_pair
        return 0
    lax.fori_loop(0, _R + 1, _init_acc, 0)

    def _init_fsrc(j, _):
        fsrc[pl.ds(j * _L, _L)] = jnp.zeros((_L,), jnp.int32)
        return 0
    lax.fori_loop(0, _CAP // _L, _init_fsrc, 0)

    def _fire_chunk(c, sb, db, csem):
        pltpu.async_copy(src_hbm.at[pl.ds(c * _CH, _CH)], sb, csem)
        pltpu.async_copy(dst_hbm.at[pl.ds(c * _CH, _CH)], db, csem)

    def _wait_chunk(c, sb, db, csem):
        pltpu.make_async_copy(src_hbm.at[pl.ds(c * _CH, _CH)], sb, csem).wait()
        pltpu.make_async_copy(dst_hbm.at[pl.ds(c * _CH, _CH)], db, csem).wait()

    def _fill_idx(idxr, b):
        for k in range(_GB // _L):
            idxr[pl.ds(k * _L, _L)] = fsrc[pl.ds(b * _GB + k * _L, _L)]

    def _fire_gather(idxr, st, gs):
        pltpu.async_copy(feat_hbm.at[idxr], st, gs)

    def _wait_gather(idxr, st, gs):
        pltpu.make_async_copy(feat_hbm.at[idxr], st, gs).wait()

    def _process(st, b):
        def _rowgrp(j, _):
            base = b * _GB + j * _L
            for k in range(_L):
                rb = plsc.load_gather(fdst, [jnp.full((_L,), base + k, jnp.int32)])
                r = j * _L + k
                for f in range(nw):
                    a_i = plsc.load_gather(acc, [rb, cols[f]])
                    a = plsc.bitcast(a_i, jnp.bfloat16)
                    v = plsc.bitcast(st[r, pl.ds(f * _L, _L)], jnp.bfloat16)
                    mx = jnp.maximum(a, v)
                    plsc.store_scatter(acc, [rb, cols[f]], plsc.bitcast(mx, jnp.int32))
            return 0
        lax.fori_loop(0, _GB // _L, _rowgrp, 0)

    def _drain(ptr):
        for k in range(_GB // _L):
            pos = ptr + k * _L + iota
            plsc.store_scatter(fsrc, [pos], jnp.zeros((_L,), jnp.int32))
            plsc.store_scatter(fdst, [pos], jnp.full((_L,), _TRASH, jnp.int32))
        nb = (ptr + _GB - 1) // _GB
        _fill_idx(idx0, 0)
        _fire_gather(idx0, stage0, gsem0)

        def _bpair(g, _):
            b0 = 2 * g
            b1 = b0 + 1

            @pl.when(b1 < nb)
            def _():
                _fill_idx(idx1, b1)
                _fire_gather(idx1, stage1, gsem1)
            _wait_gather(idx0, stage0, gsem0)
            _process(stage0, b0)

            @pl.when(b1 < nb)
            def _():
                @pl.when(b1 + 1 < nb)
                def _():
                    _fill_idx(idx0, b1 + 1)
                    _fire_gather(idx0, stage0, gsem0)
                _wait_gather(idx1, stage1, gsem1)
                _process(stage1, b1)
            return 0
        lax.fori_loop(0, (nb + 1) // 2, _bpair, 0)

    def _scan_buf(sb, db, ptr_v):
        U = 4

        def _scan(j, p_v):
            base = j * (U * _L)
            sv = [sb[pl.ds(base + u * _L, _L)] for u in range(U)]
            dv = [db[pl.ds(base + u * _L, _L)] for u in range(U)]
            ms = [(d >= lo) & (d < hi) for d in dv]
            cnt = [plsc.all_reduce_population_count(m) for m in ms]
            offs = [p_v]
            for u in range(1, U):
                offs.append(offs[-1] + cnt[u - 1])
            for u in range(U):
                mi = jnp.where(ms[u], 1, 0)
                cs = plsc.cumsum(mi)
                pos = offs[u] + cs - mi
                plsc.store_scatter(fdst, [pos], dv[u] - lo, mask=ms[u])
                plsc.store_scatter(fsrc, [pos], sv[u], mask=ms[u])
            return offs[-1] + cnt[-1]
        return lax.fori_loop(0, _CH // (U * _L), _scan, ptr_v)

    def _scan_drain(ptr_v):
        ptr = jnp.max(ptr_v)

        @pl.when(ptr >= _FLUSH)
        def _():
            _drain(ptr)
        return jnp.where(ptr >= _FLUSH, 0, ptr_v)

    _fire_chunk(0, srcb0, dstb0, csem0)

    def _pair(p, ptr_v):
        c0 = 2 * p
        _fire_chunk(c0 + 1, srcb1, dstb1, csem1)
        _wait_chunk(c0, srcb0, dstb0, csem0)
        ptr_v = _scan_buf(srcb0, dstb0, ptr_v)
        ptr_v = _scan_drain(ptr_v)

        @pl.when(c0 + 2 < _NCH)
        def _():
            _fire_chunk(c0 + 2, srcb0, dstb0, csem0)
        _wait_chunk(c0 + 1, srcb1, dstb1, csem1)
        ptr_v = _scan_buf(srcb1, dstb1, ptr_v)
        ptr_v = _scan_drain(ptr_v)
        return ptr_v
    ptr_v = lax.fori_loop(0, _NCH // 2, _pair, jnp.zeros((_L,), jnp.int32))
    ptr = jnp.max(ptr_v)

    @pl.when(ptr > 0)
    def _():
        _drain(ptr)

    def _fix(r, _):
        for f in range(nw):
            v_i = acc[r, pl.ds(f * _L, _L)]
            v = plsc.bitcast(v_i, jnp.bfloat16)
            v = jnp.where(v == jnp.bfloat16(_NEG), jnp.bfloat16(0), v)
            acc[r, pl.ds(f * _L, _L)] = plsc.bitcast(v, jnp.int32)
        return 0
    lax.fori_loop(0, _R, _fix, 0)

    @pl.when(wid < _NW - 1)
    def _():
        pltpu.sync_copy(acc.at[pl.ds(0, _R)], out_hbm.at[pl.ds(lo, _R)])

    @pl.when(wid == _NW - 1)
    def _():
        pltpu.sync_copy(acc.at[pl.ds(0, _LAST)], out_hbm.at[pl.ds(lo, _LAST)])


def _sc_segmax(feat, src, dst, D):
    mesh = plsc.VectorSubcoreMesh(core_axis_name="c", subcore_axis_name="s",
                                  num_cores=_NC, num_subcores=_NS)
    return pl.kernel(
        functools.partial(_segmax_body, D),
        out_type=jax.ShapeDtypeStruct((N, D // 2), jnp.int32),
        mesh=mesh,
        compiler_params=pltpu.CompilerParams(needs_layout_passes=False),
        scratch_types=[
            pltpu.VMEM((_CH,), jnp.int32),
            pltpu.VMEM((_CH,), jnp.int32),
            pltpu.VMEM((_CH,), jnp.int32),
            pltpu.VMEM((_CH,), jnp.int32),
            pltpu.VMEM((_CAP,), jnp.int32),
            pltpu.VMEM((_CAP,), jnp.int32),
            pltpu.VMEM((_GB,), jnp.int32),
            pltpu.VMEM((_GB,), jnp.int32),
            pltpu.VMEM((_GB, D_IN), jnp.int32),
            pltpu.VMEM((_GB, D_IN), jnp.int32),
            pltpu.VMEM((_R + 1, D // 2), jnp.int32),
            pltpu.SemaphoreType.DMA,
            pltpu.SemaphoreType.DMA,
            pltpu.SemaphoreType.DMA,
            pltpu.SemaphoreType.DMA,
        ],
    )(feat, src, dst)

_ROWS = 1024


def _dense1_body(agg_ref, x_ref, wl_ref, wr_ref, bl_ref, g_ref, be_ref, o_ref):
    agg = agg_ref[...].astype(jnp.float32)
    x = x_ref[...]
    h = (jnp.dot(agg, wl_ref[...], preferred_element_type=jnp.float32)
         + jnp.dot(x, wr_ref[...], preferred_element_type=jnp.float32)
         + bl_ref[...])
    mu = jnp.mean(h, axis=-1, keepdims=True)
    var = jnp.mean((h - mu) ** 2, axis=-1, keepdims=True)
    h = (h - mu) * jax.lax.rsqrt(var + EPS) * g_ref[...] + be_ref[...]
    h = jnp.maximum(h, 0.0)
    o_ref[...] = jnp.concatenate([h, h, h, h], axis=1).astype(jnp.bfloat16)


def _dense2_body(agg_ref, h_ref, wl_ref, wr_ref, bl_ref, g_ref, be_ref,
                 wm1_ref, bm1_ref, wm2_ref, bm2_ref, o_ref):
    agg = agg_ref[...].astype(jnp.float32)
    hp = h_ref[:, :D_H].astype(jnp.float32)
    h = (jnp.dot(agg, wl_ref[...], preferred_element_type=jnp.float32)
         + jnp.dot(hp, wr_ref[...], preferred_element_type=jnp.float32)
         + bl_ref[...])
    mu = jnp.mean(h, axis=-1, keepdims=True)
    var = jnp.mean((h - mu) ** 2, axis=-1, keepdims=True)
    h = (h - mu) * jax.lax.rsqrt(var + EPS) * g_ref[...] + be_ref[...]
    h = jnp.maximum(h, 0.0)
    m = jnp.maximum(jnp.dot(h, wm1_ref[...], preferred_element_type=jnp.float32)
                    + bm1_ref[...], 0.0)
    z = jnp.sum(m * wm2_ref[...], axis=-1, keepdims=True) + bm2_ref[...]
    o_ref[...] = jax.nn.sigmoid(z)


def _full(shape):
    return pl.BlockSpec(shape, lambda i: tuple(0 for _ in shape))


def _dense1(agg, x, wlT, wrT, bl, g, be):
    grid = (pl.cdiv(N, _ROWS),)
    return pl.pallas_call(
        _dense1_body,
        grid=grid,
        in_specs=[
            pl.BlockSpec((_ROWS, D_IN), lambda i: (i, 0)),
            pl.BlockSpec((_ROWS, D_IN), lambda i: (i, 0)),
            _full((D_IN, D_H)), _full((D_IN, D_H)),
            _full((1, D_H)), _full((1, D_H)), _full((1, D_H)),
        ],
        out_specs=pl.BlockSpec((_ROWS, 4 * D_H), lambda i: (i, 0)),
        out_shape=jax.ShapeDtypeStruct((N, 4 * D_H), jnp.bfloat16),
    )(agg, x, wlT, wrT, bl, g, be)


def _dense2(agg, h, wlT, wrT, bl, g, be, wm1T, bm1, wm2, bm2):
    grid = (pl.cdiv(N, _ROWS),)
    return pl.pallas_call(
        _dense2_body,
        grid=grid,
        in_specs=[
            pl.BlockSpec((_ROWS, D_H), lambda i: (i, 0)),
            pl.BlockSpec((_ROWS, 4 * D_H), lambda i: (i, 0)),
            _full((D_H, D_H)), _full((D_H, D_H)),
            _full((1, D_H)), _full((1, D_H)), _full((1, D_H)),
            _full((D_H, D_H // 2)), _full((1, D_H // 2)),
            _full((1, D_H // 2)), _full((1, 1)),
        ],
        out_specs=pl.BlockSpec((_ROWS, 1), lambda i: (i, 0)),
        out_shape=jax.ShapeDtypeStruct((N, 1), jnp.float32),
    )(agg, h, wlT, wrT, bl, g, be, wm1T, bm1, wm2, bm2)


def _unpack_bf16(a_i32):
    n, w = a_i32.shape
    return jax.lax.bitcast_convert_type(a_i32, jnp.bfloat16).reshape(n, 2 * w)


def _pack_i32(a_bf16):
    n, c = a_bf16.shape
    return jax.lax.bitcast_convert_type(a_bf16.reshape(n, c // 2, 2), jnp.int32)


def kernel(x, edge_index, W_l1, b_l1, W_r1, W_l2, b_l2, W_r2,
           g1, be1, g2, be2, Wm1, bm1, Wm2, bm2):
    src = edge_index[0].astype(jnp.int32)
    dst = edge_index[1].astype(jnp.int32)
    xi = _pack_i32(x.astype(jnp.bfloat16))
    feat1 = jnp.concatenate([xi, xi], axis=1)
    agg1 = _unpack_bf16(_sc_segmax(feat1, src, dst, D_IN))
    h1 = _dense1(agg1, x, W_l1.T, W_r1.T, b_l1[None, :], g1[None, :], be1[None, :])
    feat2 = _pack_i32(h1)
    agg2 = _unpack_bf16(_sc_segmax(feat2, src, dst, D_H))
    out = _dense2(agg2, h1, W_l2.T, W_r2.T, b_l2[None, :], g2[None, :], be2[None, :],
                  Wm1.T, bm1[None, :], Wm2[:1, :], bm2[None, :])
    return out[:, 0]


# speedup vs baseline: 3.7111x; 1.4916x over previous
import functools

import jax
import jax.numpy as jnp
from jax import lax
from jax.experimental import pallas as pl
from jax.experimental.pallas import tpu as pltpu
from jax.experimental.pallas import tpu_sc as plsc

N = 10000
E = 320000
D_IN = 128
D_H = 64
EPS = 1e-5


_NC, _NS, _L = 2, 16, 16
_NW = _NC * _NS
_R = 320
_LAST = N - (_NW - 1) * _R
_CH = 6400
_NCH = E // _CH
_GB = 128
_FLUSH = 15 * _GB
_CAP = _FLUSH - 1 + _CH + _GB + _L
_TRASH = _R

_NEG = float("-inf")


_LCAP = E + 4224
_BLK = 2048


def _segmax_body(D, write_lists, feat_hbm, *refs):
    if write_lists:
        (src_hbm, dst_hbm, out_hbm, lsrc_hbm, ldst_hbm, counts_hbm,
         srcb0, dstb0, srcb1, dstb1, fsrc, fdst,
         idx0, idx1, stage0, stage1, acc, cntb,
         csem0, csem1, gsem0, gsem1, lsem) = refs
    else:
        (lsrc_hbm, ldst_hbm, counts_hbm, out_hbm,
         fsrc, fdst,
         idx0, idx1, stage0, stage1, acc, cntb,
         csem0, csem1, gsem0, gsem1, lsem) = refs
    wid = lax.axis_index("s") * _NC + lax.axis_index("c")
    lo = wid * _R
    hi = lo + _R
    iota = lax.broadcasted_iota(jnp.int32, (_L,), 0)
    nw = D // 32
    cols = [iota + f * _L for f in range(nw)]
    neg_pair = plsc.bitcast(jnp.full((2 * _L,), _NEG, jnp.bfloat16), jnp.int32)

    def _init_acc(r, _):
        for f in range(nw):
            acc[r, pl.ds(f * _L, _L)] = neg_pair
        return 0
    lax.fori_loop(0, _R + 1, _init_acc, 0)

    if write_lists:
        def _init_fsrc(j, _):
            fsrc[pl.ds(j * _L, _L)] = jnp.zeros((_L,), jnp.int32)
            return 0
        lax.fori_loop(0, _CAP // _L, _init_fsrc, 0)

    def _fire_chunk(c, sb, db, csem):
        pltpu.async_copy(src_hbm.at[pl.ds(c * _CH, _CH)], sb, csem)
        pltpu.async_copy(dst_hbm.at[pl.ds(c * _CH, _CH)], db, csem)

    def _wait_chunk(c, sb, db, csem):
        pltpu.make_async_copy(src_hbm.at[pl.ds(c * _CH, _CH)], sb, csem).wait()
        pltpu.make_async_copy(dst_hbm.at[pl.ds(c * _CH, _CH)], db, csem).wait()
    if not write_lists:
        _fire_chunk = _wait_chunk = None

    def _fill_idx(idxr, b):
        for k in range(_GB // _L):
            idxr[pl.ds(k * _L, _L)] = fsrc[pl.ds(b * _GB + k * _L, _L)]

    def _fire_gather(idxr, st, gs):
        pltpu.async_copy(feat_hbm.at[idxr], st, gs)

    def _wait_gather(idxr, st, gs):
        pltpu.make_async_copy(feat_hbm.at[idxr], st, gs).wait()

    def _process(st, b):
        def _rowgrp(j, _):
            base = b * _GB + j * _L
            for k in range(_L):
                rb = plsc.load_gather(fdst, [jnp.full((_L,), base + k, jnp.int32)])
                r = j * _L + k
                for f in range(nw):
                    a_i = plsc.load_gather(acc, [rb, cols[f]])
                    a = plsc.bitcast(a_i, jnp.bfloat16)
                    v = plsc.bitcast(st[r, pl.ds(f * _L, _L)], jnp.bfloat16)
                    mx = jnp.maximum(a, v)
                    plsc.store_scatter(acc, [rb, cols[f]], plsc.bitcast(mx, jnp.int32))
            return 0
        lax.fori_loop(0, _GB // _L, _rowgrp, 0)

    def _pad_tail(ptr):
        for k in range(_GB // _L):
            pos = ptr + k * _L + iota
            plsc.store_scatter(fsrc, [pos], jnp.zeros((_L,), jnp.int32))
            plsc.store_scatter(fdst, [pos], jnp.full((_L,), _TRASH, jnp.int32))

    def _process_list(nb):
        _fill_idx(idx0, 0)
        _fire_gather(idx0, stage0, gsem0)

        def _bpair(g, _):
            b0 = 2 * g
            b1 = b0 + 1

            @pl.when(b1 < nb)
            def _():
                _fill_idx(idx1, b1)
                _fire_gather(idx1, stage1, gsem1)
            _wait_gather(idx0, stage0, gsem0)
            _process(stage0, b0)

            @pl.when(b1 < nb)
            def _():
                @pl.when(b1 + 1 < nb)
                def _():
                    _fill_idx(idx0, b1 + 1)
                    _fire_gather(idx0, stage0, gsem0)
                _wait_gather(idx1, stage1, gsem1)
                _process(stage1, b1)
            return 0
        lax.fori_loop(0, (nb + 1) // 2, _bpair, 0)

    def _scan_buf(sb, db, ptr_v):
        U = 4

        def _scan(j, p_v):
            base = j * (U * _L)
            sv = [sb[pl.ds(base + u * _L, _L)] for u in range(U)]
            dv = [db[pl.ds(base + u * _L, _L)] for u in range(U)]
            ms = [(d >= lo) & (d < hi) for d in dv]
            cnt = [plsc.all_reduce_population_count(m) for m in ms]
            offs = [p_v]
            for u in range(1, U):
                offs.append(offs[-1] + cnt[u - 1])
            for u in range(U):
                mi = jnp.where(ms[u], 1, 0)
                cs = plsc.cumsum(mi)
                pos = offs[u] + cs - mi
                plsc.store_scatter(fdst, [pos], dv[u] - lo, mask=ms[u])
                plsc.store_scatter(fsrc, [pos], sv[u], mask=ms[u])
            return offs[-1] + cnt[-1]
        return lax.fori_loop(0, _CH // (U * _L), _scan, ptr_v)

    def _scan_drain(ptr_v):
        ptr = jnp.max(ptr_v)

        @pl.when(ptr >= _FLUSH)
        def _():
            _drain(ptr)
        return jnp.where(ptr >= _FLUSH, 0, ptr_v)

    if write_lists:
        def _flush_step(carry):
            p_v, k = carry
            ptr = jnp.max(p_v)
            loff = wid * _LCAP + k * _FLUSH
            pltpu.async_copy(fsrc.at[pl.ds(0, _FLUSH)], lsrc_hbm.at[pl.ds(loff, _FLUSH)], lsem)
            pltpu.async_copy(fdst.at[pl.ds(0, _FLUSH)], ldst_hbm.at[pl.ds(loff, _FLUSH)], lsem)
            _process_list(_FLUSH // _GB)
            pltpu.make_async_copy(fsrc.at[pl.ds(0, _FLUSH)], lsrc_hbm.at[pl.ds(loff, _FLUSH)], lsem).wait()
            pltpu.make_async_copy(fdst.at[pl.ds(0, _FLUSH)], ldst_hbm.at[pl.ds(loff, _FLUSH)], lsem).wait()
            rem = ptr - _FLUSH

            def _mv(j, _):
                fsrc[pl.ds(j * _L, _L)] = fsrc[pl.ds(_FLUSH + j * _L, _L)]
                fdst[pl.ds(j * _L, _L)] = fdst[pl.ds(_FLUSH + j * _L, _L)]
                return 0
            lax.fori_loop(0, (rem + _L - 1) // _L, _mv, 0)
            return (p_v - _FLUSH, k + 1)

        def _maybe_flush(carry):
            return lax.while_loop(lambda c: jnp.max(c[0]) >= _FLUSH,
                                  _flush_step, carry)

        _fire_chunk(0, srcb0, dstb0, csem0)

        def _pair(p, carry):
            ptr_v, k = carry
            c0 = 2 * p
            _fire_chunk(c0 + 1, srcb1, dstb1, csem1)
            _wait_chunk(c0, srcb0, dstb0, csem0)
            ptr_v = _scan_buf(srcb0, dstb0, ptr_v)
            ptr_v, k = _maybe_flush((ptr_v, k))

            @pl.when(c0 + 2 < _NCH)
            def _():
                _fire_chunk(c0 + 2, srcb0, dstb0, csem0)
            _wait_chunk(c0 + 1, srcb1, dstb1, csem1)
            ptr_v = _scan_buf(srcb1, dstb1, ptr_v)
            ptr_v, k = _maybe_flush((ptr_v, k))
            return (ptr_v, k)
        ptr_v, nfl = lax.fori_loop(0, _NCH // 2, _pair,
                                   (jnp.zeros((_L,), jnp.int32), jnp.int32(0)))
        ptr = jnp.max(ptr_v)
        npad = ((ptr + _GB - 1) // _GB) * _GB

        @pl.when(ptr > 0)
        def _():
            _pad_tail(ptr)
            loff = wid * _LCAP + nfl * _FLUSH
            pltpu.async_copy(fsrc.at[pl.ds(0, _FLUSH + _GB)], lsrc_hbm.at[pl.ds(loff, _FLUSH + _GB)], lsem)
            pltpu.async_copy(fdst.at[pl.ds(0, _FLUSH + _GB)], ldst_hbm.at[pl.ds(loff, _FLUSH + _GB)], lsem)
            _process_list((ptr + _GB - 1) // _GB)
            pltpu.make_async_copy(fsrc.at[pl.ds(0, _FLUSH + _GB)], lsrc_hbm.at[pl.ds(loff, _FLUSH + _GB)], lsem).wait()
            pltpu.make_async_copy(fdst.at[pl.ds(0, _FLUSH + _GB)], ldst_hbm.at[pl.ds(loff, _FLUSH + _GB)], lsem).wait()

        total = nfl * _FLUSH + npad
        cntb[pl.ds(0, _L)] = jnp.full((_L,), 0, jnp.int32) + total
        pltpu.sync_copy(cntb, counts_hbm.at[pl.ds(wid * _L, _L)])
    else:
        pltpu.sync_copy(counts_hbm.at[pl.ds(wid * _L, _L)], cntb)
        cnt = jnp.max(cntb[pl.ds(0, _L)])
        nblk = (cnt + _BLK - 1) // _BLK

        def _blk(i, _):
            off = wid * _LCAP + i * _BLK
            pltpu.sync_copy(lsrc_hbm.at[pl.ds(off, _BLK)], fsrc.at[pl.ds(0, _BLK)])
            pltpu.sync_copy(ldst_hbm.at[pl.ds(off, _BLK)], fdst.at[pl.ds(0, _BLK)])
            n = jnp.minimum(cnt - i * _BLK, _BLK)
            _process_list((n + _GB - 1) // _GB)
            return 0
        lax.fori_loop(0, nblk, _blk, 0)

    def _fix(r, _):
        for f in range(nw):
            v_i = acc[r, pl.ds(f * _L, _L)]
            v = plsc.bitcast(v_i, jnp.bfloat16)
            v = jnp.where(v == jnp.bfloat16(_NEG), jnp.bfloat16(0), v)
            acc[r, pl.ds(f * _L, _L)] = plsc.bitcast(v, jnp.int32)
        return 0
    lax.fori_loop(0, _R, _fix, 0)

    @pl.when(wid < _NW - 1)
    def _():
        pltpu.sync_copy(acc.at[pl.ds(0, _R)], out_hbm.at[pl.ds(lo, _R)])

    @pl.when(wid == _NW - 1)
    def _():
        pltpu.sync_copy(acc.at[pl.ds(0, _LAST)], out_hbm.at[pl.ds(lo, _LAST)])


_MESH = plsc.VectorSubcoreMesh(core_axis_name="c", subcore_axis_name="s",
                               num_cores=_NC, num_subcores=_NS)
_COMMON_SCRATCH = [
    pltpu.VMEM((_CAP,), jnp.int32),
    pltpu.VMEM((_CAP,), jnp.int32),
    pltpu.VMEM((_GB,), jnp.int32),
    pltpu.VMEM((_GB,), jnp.int32),
    pltpu.VMEM((_GB, D_IN), jnp.int32),
    pltpu.VMEM((_GB, D_IN), jnp.int32),
]
_TAIL_SCRATCH = [
    pltpu.VMEM((_L,), jnp.int32),
    pltpu.SemaphoreType.DMA,
    pltpu.SemaphoreType.DMA,
    pltpu.SemaphoreType.DMA,
    pltpu.SemaphoreType.DMA,
    pltpu.SemaphoreType.DMA,
]


def _sc_segmax_write(feat, src, dst, D):
    return pl.kernel(
        functools.partial(_segmax_body, D, True),
        out_type=(jax.ShapeDtypeStruct((N, D // 2), jnp.int32),
                  jax.ShapeDtypeStruct((_NW * _LCAP,), jnp.int32),
                  jax.ShapeDtypeStruct((_NW * _LCAP,), jnp.int32),
                  jax.ShapeDtypeStruct((_NW * _L,), jnp.int32)),
        mesh=_MESH,
        compiler_params=pltpu.CompilerParams(needs_layout_passes=False),
        scratch_types=[
            pltpu.VMEM((_CH,), jnp.int32),
            pltpu.VMEM((_CH,), jnp.int32),
            pltpu.VMEM((_CH,), jnp.int32),
            pltpu.VMEM((_CH,), jnp.int32),
        ] + _COMMON_SCRATCH + [
            pltpu.VMEM((_R + 1, D // 2), jnp.int32),
        ] + _TAIL_SCRATCH,
    )(feat, src, dst)


def _sc_segmax_read(feat, lsrc, ldst, counts, D):
    return pl.kernel(
        functools.partial(_segmax_body, D, False),
        out_type=jax.ShapeDtypeStruct((N, D // 2), jnp.int32),
        mesh=_MESH,
        compiler_params=pltpu.CompilerParams(needs_layout_passes=False),
        scratch_types=_COMMON_SCRATCH + [
            pltpu.VMEM((_R + 1, D // 2), jnp.int32),
        ] + _TAIL_SCRATCH,
    )(feat, lsrc, ldst, counts)

_ROWS = 1024


def _dense1_body(agg_ref, x_ref, wl_ref, wr_ref, bl_ref, g_ref, be_ref, o_ref):
    agg = agg_ref[...].astype(jnp.float32)
    x = x_ref[...]
    h = (jnp.dot(agg, wl_ref[...], preferred_element_type=jnp.float32)
         + jnp.dot(x, wr_ref[...], preferred_element_type=jnp.float32)
         + bl_ref[...])
    mu = jnp.mean(h, axis=-1, keepdims=True)
    var = jnp.mean((h - mu) ** 2, axis=-1, keepdims=True)
    h = (h - mu) * jax.lax.rsqrt(var + EPS) * g_ref[...] + be_ref[...]
    h = jnp.maximum(h, 0.0)
    o_ref[...] = jnp.concatenate([h, h, h, h], axis=1).astype(jnp.bfloat16)


def _dense2_body(agg_ref, h_ref, wl_ref, wr_ref, bl_ref, g_ref, be_ref,
                 wm1_ref, bm1_ref, wm2_ref, bm2_ref, o_ref):
    agg = agg_ref[...].astype(jnp.float32)
    hp = h_ref[:, :D_H].astype(jnp.float32)
    h = (jnp.dot(agg, wl_ref[...], preferred_element_type=jnp.float32)
         + jnp.dot(hp, wr_ref[...], preferred_element_type=jnp.float32)
         + bl_ref[...])
    mu = jnp.mean(h, axis=-1, keepdims=True)
    var = jnp.mean((h - mu) ** 2, axis=-1, keepdims=True)
    h = (h - mu) * jax.lax.rsqrt(var + EPS) * g_ref[...] + be_ref[...]
    h = jnp.maximum(h, 0.0)
    m = jnp.maximum(jnp.dot(h, wm1_ref[...], preferred_element_type=jnp.float32)
                    + bm1_ref[...], 0.0)
    z = jnp.sum(m * wm2_ref[...], axis=-1, keepdims=True) + bm2_ref[...]
    o_ref[...] = jax.nn.sigmoid(z)


def _full(shape):
    return pl.BlockSpec(shape, lambda i: tuple(0 for _ in shape))


def _dense1(agg, x, wlT, wrT, bl, g, be):
    grid = (pl.cdiv(N, _ROWS),)
    return pl.pallas_call(
        _dense1_body,
        grid=grid,
        in_specs=[
            pl.BlockSpec((_ROWS, D_IN), lambda i: (i, 0)),
            pl.BlockSpec((_ROWS, D_IN), lambda i: (i, 0)),
            _full((D_IN, D_H)), _full((D_IN, D_H)),
            _full((1, D_H)), _full((1, D_H)), _full((1, D_H)),
        ],
        out_specs=pl.BlockSpec((_ROWS, 4 * D_H), lambda i: (i, 0)),
        out_shape=jax.ShapeDtypeStruct((N, 4 * D_H), jnp.bfloat16),
    )(agg, x, wlT, wrT, bl, g, be)


def _dense2(agg, h, wlT, wrT, bl, g, be, wm1T, bm1, wm2, bm2):
    grid = (pl.cdiv(N, _ROWS),)
    return pl.pallas_call(
        _dense2_body,
        grid=grid,
        in_specs=[
            pl.BlockSpec((_ROWS, D_H), lambda i: (i, 0)),
            pl.BlockSpec((_ROWS, 4 * D_H), lambda i: (i, 0)),
            _full((D_H, D_H)), _full((D_H, D_H)),
            _full((1, D_H)), _full((1, D_H)), _full((1, D_H)),
            _full((D_H, D_H // 2)), _full((1, D_H // 2)),
            _full((1, D_H // 2)), _full((1, 1)),
        ],
        out_specs=pl.BlockSpec((_ROWS, 1), lambda i: (i, 0)),
        out_shape=jax.ShapeDtypeStruct((N, 1), jnp.float32),
    )(agg, h, wlT, wrT, bl, g, be, wm1T, bm1, wm2, bm2)


def _unpack_bf16(a_i32):
    n, w = a_i32.shape
    return jax.lax.bitcast_convert_type(a_i32, jnp.bfloat16).reshape(n, 2 * w)


def _pack_i32(a_bf16):
    n, c = a_bf16.shape
    return jax.lax.bitcast_convert_type(a_bf16.reshape(n, c // 2, 2), jnp.int32)


def kernel(x, edge_index, W_l1, b_l1, W_r1, W_l2, b_l2, W_r2,
           g1, be1, g2, be2, Wm1, bm1, Wm2, bm2):
    src = edge_index[0].astype(jnp.int32)
    dst = edge_index[1].astype(jnp.int32)
    xi = _pack_i32(x.astype(jnp.bfloat16))
    feat1 = jnp.concatenate([xi, xi], axis=1)
    agg1_p, lsrc, ldst, counts = _sc_segmax_write(feat1, src, dst, D_IN)
    agg1 = _unpack_bf16(agg1_p)
    h1 = _dense1(agg1, x, W_l1.T, W_r1.T, b_l1[None, :], g1[None, :], be1[None, :])
    feat2 = _pack_i32(h1)
    agg2 = _unpack_bf16(_sc_segmax_read(feat2, lsrc, ldst, counts, D_H))
    out = _dense2(agg2, h1, W_l2.T, W_r2.T, b_l2[None, :], g2[None, :], be2[None, :],
                  Wm1.T, bm1[None, :], Wm2[:1, :], bm2[None, :])
    return out[:, 0]


# speedup vs baseline: 3.8160x; 1.0283x over previous
import functools

import jax
import jax.numpy as jnp
from jax import lax
from jax.experimental import pallas as pl
from jax.experimental.pallas import tpu as pltpu
from jax.experimental.pallas import tpu_sc as plsc

N = 10000
E = 320000
D_IN = 128
D_H = 64
EPS = 1e-5


_NC, _NS, _L = 2, 16, 16
_NW = _NC * _NS
_R = 320
_LAST = N - (_NW - 1) * _R
_CH = 6400
_NCH = E // _CH
_GB = 128
_FLUSH = 15 * _GB
_CAP = _FLUSH - 1 + _CH + _GB + _L
_TRASH = _R

_NEG = float("-inf")


_LCAP = E + 4224
_BLK = 2048


def _segmax_body(D, write_lists, feat_hbm, *refs):
    if write_lists:
        (src_hbm, dst_hbm, out_hbm, lsrc_hbm, ldst_hbm, counts_hbm,
         srcb0, dstb0, srcb1, dstb1, fsrc, fdst,
         idx0, idx1, stage0, stage1, acc, cntb,
         csem0, csem1, gsem0, gsem1, lsem) = refs
    else:
        (lsrc_hbm, ldst_hbm, counts_hbm, out_hbm,
         fsrc, fdst,
         idx0, idx1, stage0, stage1, acc, cntb,
         csem0, csem1, gsem0, gsem1, lsem) = refs
    wid = lax.axis_index("s") * _NC + lax.axis_index("c")
    lo = wid * _R
    hi = lo + _R
    iota = lax.broadcasted_iota(jnp.int32, (_L,), 0)
    nw = D // 32
    cols = [iota + f * _L for f in range(nw)]
    neg_pair = plsc.bitcast(jnp.full((2 * _L,), _NEG, jnp.bfloat16), jnp.int32)

    def _init_acc(r, _):
        for f in range(nw):
            acc[r, pl.ds(f * _L, _L)] = neg_pair
        return 0
    lax.fori_loop(0, _R + 1, _init_acc, 0)

    if write_lists:
        def _init_fsrc(j, _):
            fsrc[pl.ds(j * _L, _L)] = jnp.zeros((_L,), jnp.int32)
            return 0
        lax.fori_loop(0, _CAP // _L, _init_fsrc, 0)

    def _fire_chunk(c, sb, db, csem):
        pltpu.async_copy(src_hbm.at[pl.ds(c * _CH, _CH)], sb, csem)
        pltpu.async_copy(dst_hbm.at[pl.ds(c * _CH, _CH)], db, csem)

    def _wait_chunk(c, sb, db, csem):
        pltpu.make_async_copy(src_hbm.at[pl.ds(c * _CH, _CH)], sb, csem).wait()
        pltpu.make_async_copy(dst_hbm.at[pl.ds(c * _CH, _CH)], db, csem).wait()
    if not write_lists:
        _fire_chunk = _wait_chunk = None

    def _fill_idx(idxr, b):
        for k in range(_GB // _L):
            idxr[pl.ds(k * _L, _L)] = fsrc[pl.ds(b * _GB + k * _L, _L)]

    def _fire_gather(idxr, st, gs):
        pltpu.async_copy(feat_hbm.at[idxr], st, gs)

    def _wait_gather(idxr, st, gs):
        pltpu.make_async_copy(feat_hbm.at[idxr], st, gs).wait()

    def _process(st, b):
        def _rowgrp(j, _):
            base = b * _GB + j * _L
            for k in range(_L):
                rb = plsc.load_gather(fdst, [jnp.full((_L,), base + k, jnp.int32)])
                r = j * _L + k
                for f in range(nw):
                    a_i = plsc.load_gather(acc, [rb, cols[f]])
                    a = plsc.bitcast(a_i, jnp.bfloat16)
                    v = plsc.bitcast(st[r, pl.ds(f * _L, _L)], jnp.bfloat16)
                    mx = jnp.maximum(a, v)
                    plsc.store_scatter(acc, [rb, cols[f]], plsc.bitcast(mx, jnp.int32))
            return 0
        lax.fori_loop(0, _GB // _L, _rowgrp, 0)

    def _pad_tail(ptr):
        for k in range(_GB // _L):
            pos = ptr + k * _L + iota
            plsc.store_scatter(fsrc, [pos], jnp.zeros((_L,), jnp.int32))
            plsc.store_scatter(fdst, [pos], jnp.full((_L,), _TRASH, jnp.int32))

    def _process_list(nb):
        _fill_idx(idx0, 0)
        _fire_gather(idx0, stage0, gsem0)

        def _bpair(g, _):
            b0 = 2 * g
            b1 = b0 + 1

            @pl.when(b1 < nb)
            def _():
                _fill_idx(idx1, b1)
                _fire_gather(idx1, stage1, gsem1)
            _wait_gather(idx0, stage0, gsem0)
            _process(stage0, b0)

            @pl.when(b1 < nb)
            def _():
                @pl.when(b1 + 1 < nb)
                def _():
                    _fill_idx(idx0, b1 + 1)
                    _fire_gather(idx0, stage0, gsem0)
                _wait_gather(idx1, stage1, gsem1)
                _process(stage1, b1)
            return 0
        lax.fori_loop(0, (nb + 1) // 2, _bpair, 0)

    def _scan_buf(sb, db, ptr_v):
        U = 8

        def _scan(j, p_v):
            base = j * (U * _L)
            sv = [sb[pl.ds(base + u * _L, _L)] for u in range(U)]
            bv = [db[pl.ds(base + u * _L, _L)] - lo for u in range(U)]
            ms = [plsc.bitcast(b, jnp.uint32) < jnp.uint32(_R) for b in bv]
            cnt = [plsc.all_reduce_population_count(m) for m in ms]
            offs = [p_v]
            for u in range(1, U):
                offs.append(offs[-1] + cnt[u - 1])
            for u in range(U):
                mi = jnp.where(ms[u], 1, 0)
                cs = plsc.cumsum(mi)
                pos = offs[u] + cs - mi
                plsc.store_scatter(fdst, [pos], bv[u], mask=ms[u])
                plsc.store_scatter(fsrc, [pos], sv[u], mask=ms[u])
            return offs[-1] + cnt[-1]
        return lax.fori_loop(0, _CH // (U * _L), _scan, ptr_v)

    def _scan_drain(ptr_v):
        ptr = jnp.max(ptr_v)

        @pl.when(ptr >= _FLUSH)
        def _():
            _drain(ptr)
        return jnp.where(ptr >= _FLUSH, 0, ptr_v)

    if write_lists:
        def _flush_step(carry):
            p_v, k = carry
            ptr = jnp.max(p_v)
            loff = wid * _LCAP + k * _FLUSH
            pltpu.async_copy(fsrc.at[pl.ds(0, _FLUSH)], lsrc_hbm.at[pl.ds(loff, _FLUSH)], lsem)
            pltpu.async_copy(fdst.at[pl.ds(0, _FLUSH)], ldst_hbm.at[pl.ds(loff, _FLUSH)], lsem)
            _process_list(_FLUSH // _GB)
            pltpu.make_async_copy(fsrc.at[pl.ds(0, _FLUSH)], lsrc_hbm.at[pl.ds(loff, _FLUSH)], lsem).wait()
            pltpu.make_async_copy(fdst.at[pl.ds(0, _FLUSH)], ldst_hbm.at[pl.ds(loff, _FLUSH)], lsem).wait()
            rem = ptr - _FLUSH

            def _mv(j, _):
                fsrc[pl.ds(j * _L, _L)] = fsrc[pl.ds(_FLUSH + j * _L, _L)]
                fdst[pl.ds(j * _L, _L)] = fdst[pl.ds(_FLUSH + j * _L, _L)]
                return 0
            lax.fori_loop(0, (rem + _L - 1) // _L, _mv, 0)
            return (p_v - _FLUSH, k + 1)

        def _maybe_flush(carry):
            return lax.while_loop(lambda c: jnp.max(c[0]) >= _FLUSH,
                                  _flush_step, carry)

        _fire_chunk(0, srcb0, dstb0, csem0)

        def _pair(p, carry):
            ptr_v, k = carry
            c0 = 2 * p
            _fire_chunk(c0 + 1, srcb1, dstb1, csem1)
            _wait_chunk(c0, srcb0, dstb0, csem0)
            ptr_v = _scan_buf(srcb0, dstb0, ptr_v)
            ptr_v, k = _maybe_flush((ptr_v, k))

            @pl.when(c0 + 2 < _NCH)
            def _():
                _fire_chunk(c0 + 2, srcb0, dstb0, csem0)
            _wait_chunk(c0 + 1, srcb1, dstb1, csem1)
            ptr_v = _scan_buf(srcb1, dstb1, ptr_v)
            ptr_v, k = _maybe_flush((ptr_v, k))
            return (ptr_v, k)
        ptr_v, nfl = lax.fori_loop(0, _NCH // 2, _pair,
                                   (jnp.zeros((_L,), jnp.int32), jnp.int32(0)))
        ptr = jnp.max(ptr_v)
        npad = ((ptr + _GB - 1) // _GB) * _GB

        @pl.when(ptr > 0)
        def _():
            _pad_tail(ptr)
            loff = wid * _LCAP + nfl * _FLUSH
            pltpu.async_copy(fsrc.at[pl.ds(0, _FLUSH + _GB)], lsrc_hbm.at[pl.ds(loff, _FLUSH + _GB)], lsem)
            pltpu.async_copy(fdst.at[pl.ds(0, _FLUSH + _GB)], ldst_hbm.at[pl.ds(loff, _FLUSH + _GB)], lsem)
            _process_list((ptr + _GB - 1) // _GB)
            pltpu.make_async_copy(fsrc.at[pl.ds(0, _FLUSH + _GB)], lsrc_hbm.at[pl.ds(loff, _FLUSH + _GB)], lsem).wait()
            pltpu.make_async_copy(fdst.at[pl.ds(0, _FLUSH + _GB)], ldst_hbm.at[pl.ds(loff, _FLUSH + _GB)], lsem).wait()

        total = nfl * _FLUSH + npad
        cntb[pl.ds(0, _L)] = jnp.full((_L,), 0, jnp.int32) + total
        pltpu.sync_copy(cntb, counts_hbm.at[pl.ds(wid * _L, _L)])
    else:
        pltpu.sync_copy(counts_hbm.at[pl.ds(wid * _L, _L)], cntb)
        cnt = jnp.max(cntb[pl.ds(0, _L)])
        nblk = (cnt + _BLK - 1) // _BLK

        def _blk(i, _):
            off = wid * _LCAP + i * _BLK
            pltpu.sync_copy(lsrc_hbm.at[pl.ds(off, _BLK)], fsrc.at[pl.ds(0, _BLK)])
            pltpu.sync_copy(ldst_hbm.at[pl.ds(off, _BLK)], fdst.at[pl.ds(0, _BLK)])
            n = jnp.minimum(cnt - i * _BLK, _BLK)
            _process_list((n + _GB - 1) // _GB)
            return 0
        lax.fori_loop(0, nblk, _blk, 0)

    def _fix(r, _):
        for f in range(nw):
            v_i = acc[r, pl.ds(f * _L, _L)]
            v = plsc.bitcast(v_i, jnp.bfloat16)
            v = jnp.where(v == jnp.bfloat16(_NEG), jnp.bfloat16(0), v)
            acc[r, pl.ds(f * _L, _L)] = plsc.bitcast(v, jnp.int32)
        return 0
    lax.fori_loop(0, _R, _fix, 0)

    @pl.when(wid < _NW - 1)
    def _():
        pltpu.sync_copy(acc.at[pl.ds(0, _R)], out_hbm.at[pl.ds(lo, _R)])

    @pl.when(wid == _NW - 1)
    def _():
        pltpu.sync_copy(acc.at[pl.ds(0, _LAST)], out_hbm.at[pl.ds(lo, _LAST)])


_MESH = plsc.VectorSubcoreMesh(core_axis_name="c", subcore_axis_name="s",
                               num_cores=_NC, num_subcores=_NS)
_COMMON_SCRATCH = [
    pltpu.VMEM((_CAP,), jnp.int32),
    pltpu.VMEM((_CAP,), jnp.int32),
    pltpu.VMEM((_GB,), jnp.int32),
    pltpu.VMEM((_GB,), jnp.int32),
    pltpu.VMEM((_GB, D_IN), jnp.int32),
    pltpu.VMEM((_GB, D_IN), jnp.int32),
]
_TAIL_SCRATCH = [
    pltpu.VMEM((_L,), jnp.int32),
    pltpu.SemaphoreType.DMA,
    pltpu.SemaphoreType.DMA,
    pltpu.SemaphoreType.DMA,
    pltpu.SemaphoreType.DMA,
    pltpu.SemaphoreType.DMA,
]


def _sc_segmax_write(feat, src, dst, D):
    return pl.kernel(
        functools.partial(_segmax_body, D, True),
        out_type=(jax.ShapeDtypeStruct((N, D // 2), jnp.int32),
                  jax.ShapeDtypeStruct((_NW * _LCAP,), jnp.int32),
                  jax.ShapeDtypeStruct((_NW * _LCAP,), jnp.int32),
                  jax.ShapeDtypeStruct((_NW * _L,), jnp.int32)),
        mesh=_MESH,
        compiler_params=pltpu.CompilerParams(needs_layout_passes=False),
        scratch_types=[
            pltpu.VMEM((_CH,), jnp.int32),
            pltpu.VMEM((_CH,), jnp.int32),
            pltpu.VMEM((_CH,), jnp.int32),
            pltpu.VMEM((_CH,), jnp.int32),
        ] + _COMMON_SCRATCH + [
            pltpu.VMEM((_R + 1, D // 2), jnp.int32),
        ] + _TAIL_SCRATCH,
    )(feat, src, dst)


def _sc_segmax_read(feat, lsrc, ldst, counts, D):
    return pl.kernel(
        functools.partial(_segmax_body, D, False),
        out_type=jax.ShapeDtypeStruct((N, D // 2), jnp.int32),
        mesh=_MESH,
        compiler_params=pltpu.CompilerParams(needs_layout_passes=False),
        scratch_types=_COMMON_SCRATCH + [
            pltpu.VMEM((_R + 1, D // 2), jnp.int32),
        ] + _TAIL_SCRATCH,
    )(feat, lsrc, ldst, counts)

_ROWS = 1024


def _dense1_body(agg_ref, x_ref, wl_ref, wr_ref, bl_ref, g_ref, be_ref, o_ref):
    agg = agg_ref[...].astype(jnp.float32)
    x = x_ref[...]
    h = (jnp.dot(agg, wl_ref[...], preferred_element_type=jnp.float32)
         + jnp.dot(x, wr_ref[...], preferred_element_type=jnp.float32)
         + bl_ref[...])
    mu = jnp.mean(h, axis=-1, keepdims=True)
    var = jnp.mean((h - mu) ** 2, axis=-1, keepdims=True)
    h = (h - mu) * jax.lax.rsqrt(var + EPS) * g_ref[...] + be_ref[...]
    h = jnp.maximum(h, 0.0)
    o_ref[...] = jnp.concatenate([h, h, h, h], axis=1).astype(jnp.bfloat16)


def _dense2_body(agg_ref, h_ref, wl_ref, wr_ref, bl_ref, g_ref, be_ref,
                 wm1_ref, bm1_ref, wm2_ref, bm2_ref, o_ref):
    agg = agg_ref[...].astype(jnp.float32)
    hp = h_ref[:, :D_H].astype(jnp.float32)
    h = (jnp.dot(agg, wl_ref[...], preferred_element_type=jnp.float32)
         + jnp.dot(hp, wr_ref[...], preferred_element_type=jnp.float32)
         + bl_ref[...])
    mu = jnp.mean(h, axis=-1, keepdims=True)
    var = jnp.mean((h - mu) ** 2, axis=-1, keepdims=True)
    h = (h - mu) * jax.lax.rsqrt(var + EPS) * g_ref[...] + be_ref[...]
    h = jnp.maximum(h, 0.0)
    m = jnp.maximum(jnp.dot(h, wm1_ref[...], preferred_element_type=jnp.float32)
                    + bm1_ref[...], 0.0)
    z = jnp.sum(m * wm2_ref[...], axis=-1, keepdims=True) + bm2_ref[...]
    o_ref[...] = jax.nn.sigmoid(z)


def _full(shape):
    return pl.BlockSpec(shape, lambda i: tuple(0 for _ in shape))


def _dense1(agg, x, wlT, wrT, bl, g, be):
    grid = (pl.cdiv(N, _ROWS),)
    return pl.pallas_call(
        _dense1_body,
        grid=grid,
        in_specs=[
            pl.BlockSpec((_ROWS, D_IN), lambda i: (i, 0)),
            pl.BlockSpec((_ROWS, D_IN), lambda i: (i, 0)),
            _full((D_IN, D_H)), _full((D_IN, D_H)),
            _full((1, D_H)), _full((1, D_H)), _full((1, D_H)),
        ],
        out_specs=pl.BlockSpec((_ROWS, 4 * D_H), lambda i: (i, 0)),
        out_shape=jax.ShapeDtypeStruct((N, 4 * D_H), jnp.bfloat16),
    )(agg, x, wlT, wrT, bl, g, be)


def _dense2(agg, h, wlT, wrT, bl, g, be, wm1T, bm1, wm2, bm2):
    grid = (pl.cdiv(N, _ROWS),)
    return pl.pallas_call(
        _dense2_body,
        grid=grid,
        in_specs=[
            pl.BlockSpec((_ROWS, D_H), lambda i: (i, 0)),
            pl.BlockSpec((_ROWS, 4 * D_H), lambda i: (i, 0)),
            _full((D_H, D_H)), _full((D_H, D_H)),
            _full((1, D_H)), _full((1, D_H)), _full((1, D_H)),
            _full((D_H, D_H // 2)), _full((1, D_H // 2)),
            _full((1, D_H // 2)), _full((1, 1)),
        ],
        out_specs=pl.BlockSpec((_ROWS, 1), lambda i: (i, 0)),
        out_shape=jax.ShapeDtypeStruct((N, 1), jnp.float32),
    )(agg, h, wlT, wrT, bl, g, be, wm1T, bm1, wm2, bm2)


def _unpack_bf16(a_i32):
    n, w = a_i32.shape
    return jax.lax.bitcast_convert_type(a_i32, jnp.bfloat16).reshape(n, 2 * w)


def _pack_i32(a_bf16):
    n, c = a_bf16.shape
    return jax.lax.bitcast_convert_type(a_bf16.reshape(n, c // 2, 2), jnp.int32)


def kernel(x, edge_index, W_l1, b_l1, W_r1, W_l2, b_l2, W_r2,
           g1, be1, g2, be2, Wm1, bm1, Wm2, bm2):
    src = edge_index[0].astype(jnp.int32)
    dst = edge_index[1].astype(jnp.int32)
    xi = _pack_i32(x.astype(jnp.bfloat16))
    feat1 = jnp.concatenate([xi, xi], axis=1)
    agg1_p, lsrc, ldst, counts = _sc_segmax_write(feat1, src, dst, D_IN)
    agg1 = _unpack_bf16(agg1_p)
    h1 = _dense1(agg1, x, W_l1.T, W_r1.T, b_l1[None, :], g1[None, :], be1[None, :])
    feat2 = _pack_i32(h1)
    agg2 = _unpack_bf16(_sc_segmax_read(feat2, lsrc, ldst, counts, D_H))
    out = _dense2(agg2, h1, W_l2.T, W_r2.T, b_l2[None, :], g2[None, :], be2[None, :],
                  Wm1.T, bm1[None, :], Wm2[:1, :], bm2[None, :])
    return out[:, 0]
